# Initial kernel scaffold; baseline (speedup 1.0000x reference)
import functools

import jax
import jax.numpy as jnp
from jax import lax
from jax.experimental import pallas as pl
from jax.experimental.pallas import tpu as pltpu
from jax.experimental.pallas import tpu_sc as plsc

F32 = jnp.float32
_BN = 256
_WIN = 128
_NSC = 2
_NTILES = 16


def _tc_project(x_p, Wf, al, ar):
    Np, F = x_p.shape
    H, D = al.shape
    nb = Np // _BN

    def body(x_ref, w_ref, al_ref, ar_ref, f_ref, el_ref, er_ref):
        fb = jnp.dot(x_ref[...], w_ref[...], preferred_element_type=F32)
        f_ref[0] = fb
        el_ref[0] = jnp.sum(fb * al_ref[...], axis=1, keepdims=True)
        er_ref[0] = jnp.sum(fb * ar_ref[...], axis=1, keepdims=True)

    return pl.pallas_call(
        body,
        grid=(H, nb),
        in_specs=[
            pl.BlockSpec((_BN, F), lambda h, i: (i, 0)),
            pl.BlockSpec((F, D), lambda h, i: (0, h)),
            pl.BlockSpec((1, D), lambda h, i: (h, 0)),
            pl.BlockSpec((1, D), lambda h, i: (h, 0)),
        ],
        out_specs=[
            pl.BlockSpec((1, _BN, D), lambda h, i: (h, i, 0)),
            pl.BlockSpec((1, _BN, 1), lambda h, i: (h, i, 0)),
            pl.BlockSpec((1, _BN, 1), lambda h, i: (h, i, 0)),
        ],
        out_shape=[
            jax.ShapeDtypeStruct((H, Np, D), F32),
            jax.ShapeDtypeStruct((H, Np, 1), F32),
            jax.ShapeDtypeStruct((H, Np, 1), F32),
        ],
    )(x_p, Wf, al, ar)


def _sc_aggregate(src_p, dst_p, el_f, er_f, feat_f, Np, D, hps, edges_all):
    Ep = src_p.shape[0]
    nslab = _NSC * hps
    rpt = Np // _NTILES
    ept = Ep // _NTILES if edges_all else Ep // (_NTILES * _NSC)
    nwin = ept // _WIN
    zrows = jnp.zeros((rpt, D), F32)
    zden = jnp.zeros((hps * rpt,), F32)
    mesh = plsc.VectorSubcoreMesh(core_axis_name="c", subcore_axis_name="s")

    @functools.partial(
        pl.kernel,
        out_type=[
            jax.ShapeDtypeStruct((nslab, Np, D), F32),
            jax.ShapeDtypeStruct((nslab, Np), F32),
        ],
        mesh=mesh,
        scratch_types=[
            pltpu.VMEM_SHARED((Np, D), F32),
            pltpu.VMEM_SHARED((hps * Np,), F32),
            pltpu.VMEM((_WIN,), jnp.int32),
            pltpu.VMEM((1, _WIN), jnp.int32),
            pltpu.VMEM((1, _WIN), jnp.int32),
            pltpu.VMEM((_WIN,), jnp.int32),
            pltpu.VMEM((_WIN,), jnp.int32),
            pltpu.VMEM((_WIN,), F32),
            pltpu.VMEM((_WIN,), F32),
            pltpu.VMEM((_WIN,), F32),
            pltpu.VMEM((_WIN, D), F32),
        ],
    )
    def k(src_h, dst_h, el_h, er_h, feat_h, zr_h, zd_h, out_h, den_h,
          out_sp, den_sp, src_v, dstr_v, dstd_v, srca_v, era_v,
          els_v, erd_v, ee_v, feat_v):
        c = lax.axis_index("c")
        s = lax.axis_index("s")
        pltpu.sync_copy(zr_h, out_sp.at[pl.ds(s * rpt, rpt)])
        pltpu.sync_copy(zd_h, den_sp.at[pl.ds(s * (hps * rpt), hps * rpt)])
        plsc.subcore_barrier()
        for p in range(hps):
            head = (c * hps + p) if edges_all else p
            hN = head * Np
            tile_base = (s * ept) if edges_all else (c * _NTILES + s) * ept

            @pl.loop(0, nwin)
            def _win(w):
                base = tile_base + w * _WIN
                pltpu.sync_copy(src_h.at[pl.ds(base, _WIN)], src_v)
                pltpu.sync_copy(dst_h.at[pl.ds(base, _WIN)], dstr_v.at[0])
                for kk in range(_WIN // 16):
                    sl = pl.ds(kk * 16, 16)
                    sv = src_v[sl]
                    dv = dstr_v[0, sl]
                    srca_v[sl] = sv + hN
                    era_v[sl] = dv + hN
                    dstd_v[0, sl] = dv + (p * Np)
                pltpu.sync_copy(el_h.at[srca_v], els_v)
                pltpu.sync_copy(er_h.at[era_v], erd_v)
                for kk in range(_WIN // 16):
                    sl = pl.ds(kk * 16, 16)
                    e = els_v[sl] + erd_v[sl]
                    e = jnp.where(e >= 0.0, e, 0.2 * e)
                    ee_v[sl] = jnp.exp(e)
                pltpu.sync_copy(ee_v, den_sp.at[dstd_v.at[0]], add=True)
                pltpu.sync_copy(feat_h.at[srca_v], feat_v)

                @pl.loop(0, _WIN)
                def _edge(ei):
                    eev = plsc.load_gather(
                        ee_v, [jnp.full((16,), ei, jnp.int32)])
                    for k2 in range(D // 16):
                        sl2 = pl.ds(k2 * 16, 16)
                        feat_v[ei, sl2] = feat_v[ei, sl2] * eev

                pltpu.sync_copy(feat_v, out_sp.at[dstr_v.at[0]], add=True)

            plsc.subcore_barrier()
            slab = c * hps + p
            pltpu.sync_copy(out_sp.at[pl.ds(s * rpt, rpt)],
                            out_h.at[slab, pl.ds(s * rpt, rpt)])
            if p < hps - 1:
                pltpu.sync_copy(zr_h, out_sp.at[pl.ds(s * rpt, rpt)])
                plsc.subcore_barrier()
        for p in range(hps):
            pltpu.sync_copy(den_sp.at[pl.ds(p * Np + s * rpt, rpt)],
                            den_h.at[c * hps + p, pl.ds(s * rpt, rpt)])

    return k(src_p, dst_p, el_f, er_f, feat_f, zrows, zden)


def _tc_norm_stats(out_slabs, den_slabs, b, n_real, sum_parts):
    S, Np, D = out_slabs.shape
    H = b.shape[0]
    HD = H * D
    nb = Np // _BN

    def body(o_ref, d_ref, b_ref, val_ref, st_ref):
        i = pl.program_id(0)
        if sum_parts:
            acc = o_ref[0]
            den = d_ref[0]
            for t in range(1, S):
                acc = acc + o_ref[t]
                den = den + d_ref[t]
            val = acc / (den + 1e-9) + b_ref[0][None, :]
        else:
            cols = []
            for hh in range(H):
                cols.append(o_ref[hh] / (d_ref[hh] + 1e-9)
                            + b_ref[hh][None, :])
            val = jnp.concatenate(cols, axis=1)
        rows = i * _BN + lax.broadcasted_iota(jnp.int32, (_BN, 1), 0)
        val = jnp.where(rows < n_real, val, 0.0)
        val_ref[...] = val

        @pl.when(i == 0)
        def _():
            st_ref[...] = jnp.zeros_like(st_ref)

        st_ref[0:1, :] += jnp.sum(val, axis=0, keepdims=True)
        st_ref[1:2, :] += jnp.sum(val * val, axis=0, keepdims=True)

    return pl.pallas_call(
        body,
        grid=(nb,),
        in_specs=[
            pl.BlockSpec((S, _BN, D), lambda i: (0, i, 0)),
            pl.BlockSpec((S, _BN, 1), lambda i: (0, i, 0)),
            pl.BlockSpec((H, D), lambda i: (0, 0)),
        ],
        out_specs=[
            pl.BlockSpec((_BN, HD), lambda i: (i, 0)),
            pl.BlockSpec((8, HD), lambda i: (0, 0)),
        ],
        out_shape=[
            jax.ShapeDtypeStruct((Np, HD), F32),
            jax.ShapeDtypeStruct((8, HD), F32),
        ],
    )(out_slabs, den_slabs, b)


def _tc_bn_relu(val, st, g, be, n_real):
    Np, HD = val.shape
    nb = Np // _BN

    def body(v_ref, st_ref, g_ref, be_ref, o_ref):
        i = pl.program_id(0)
        mean = st_ref[0:1, :] / n_real
        var = st_ref[1:2, :] / n_real - mean * mean
        inv = lax.rsqrt(var + 1e-5)
        hn = (v_ref[...] - mean) * inv * g_ref[...] + be_ref[...]
        hn = jnp.maximum(hn, 0.0)
        rows = i * _BN + lax.broadcasted_iota(jnp.int32, (_BN, 1), 0)
        o_ref[...] = jnp.where(rows < n_real, hn, 0.0)

    return pl.pallas_call(
        body,
        grid=(nb,),
        in_specs=[
            pl.BlockSpec((_BN, HD), lambda i: (i, 0)),
            pl.BlockSpec((8, HD), lambda i: (0, 0)),
            pl.BlockSpec((1, HD), lambda i: (0, 0)),
            pl.BlockSpec((1, HD), lambda i: (0, 0)),
        ],
        out_specs=pl.BlockSpec((_BN, HD), lambda i: (i, 0)),
        out_shape=jax.ShapeDtypeStruct((Np, HD), F32),
    )(val, st, g, be)


def _tc_head(h2, Wout, bout):
    Np, D = h2.shape
    C = Wout.shape[1]
    nb = Np // _BN

    def body(h_ref, w_ref, bo_ref, f_ref, o_ref):
        hb = h_ref[...]
        n2 = jnp.sum(hb * hb, axis=1, keepdims=True)
        nr = jnp.maximum(jnp.sqrt(n2), 1e-12)
        ft = hb / nr
        f_ref[...] = ft
        o_ref[...] = jnp.dot(ft, w_ref[...],
                             preferred_element_type=F32) + bo_ref[...]

    return pl.pallas_call(
        body,
        grid=(nb,),
        in_specs=[
            pl.BlockSpec((_BN, D), lambda i: (i, 0)),
            pl.BlockSpec((D, C), lambda i: (0, 0)),
            pl.BlockSpec((1, C), lambda i: (0, 0)),
        ],
        out_specs=[
            pl.BlockSpec((_BN, D), lambda i: (i, 0)),
            pl.BlockSpec((_BN, C), lambda i: (i, 0)),
        ],
        out_shape=[
            jax.ShapeDtypeStruct((Np, D), F32),
            jax.ShapeDtypeStruct((Np, C), F32),
        ],
    )(h2, Wout, bout)


def kernel(x, edge_index_0, edge_index_1, W0, al0, ar0, b0, g0, be0,
           W1, al1, ar1, b1, g1, be1, Wout, bout):
    N, F = x.shape
    H, D = al0.shape
    C = Wout.shape[1]
    E = edge_index_0.shape[1]

    Np = ((N + 64 + _BN - 1) // _BN) * _BN
    egran = _NSC * _NTILES * _WIN
    Ep = ((E + egran - 1) // egran) * egran
    pad_idx = N + (jnp.arange(Ep - E, dtype=jnp.int32) % 64)

    def pad_edges(ei):
        return (jnp.concatenate([ei[0], pad_idx]),
                jnp.concatenate([ei[1], pad_idx]))

    x_p = jnp.zeros((Np, F), F32).at[:N].set(x)
    src0, dst0 = pad_edges(edge_index_0)
    src1, dst1 = pad_edges(edge_index_1)

    feat0, el0, er0 = _tc_project(x_p, W0.reshape(F, H * D), al0, ar0)
    out0, den0 = _sc_aggregate(src0, dst0, el0.reshape(-1), er0.reshape(-1),
                               feat0.reshape(H * Np, D), Np, D,
                               hps=H // _NSC, edges_all=True)
    val0, st0 = _tc_norm_stats(out0, den0.reshape(H, Np, 1), b0, N,
                               sum_parts=False)
    h1 = _tc_bn_relu(val0, st0, g0.reshape(1, H * D), be0.reshape(1, H * D), N)

    feat1, el1, er1 = _tc_project(h1, W1.reshape(H * D, D), al1, ar1)
    out1, den1 = _sc_aggregate(src1, dst1, el1.reshape(-1), er1.reshape(-1),
                               feat1.reshape(Np, D), Np, D,
                               hps=1, edges_all=False)
    val1, st1 = _tc_norm_stats(out1, den1.reshape(_NSC, Np, 1), b1, N,
                               sum_parts=True)
    h2 = _tc_bn_relu(val1, st1, g1.reshape(1, D), be1.reshape(1, D), N)

    feat_out, logits = _tc_head(h2, Wout, bout.reshape(1, C))
    return (logits[:N], feat_out[:N])


# speedup vs baseline: 18.1797x; 18.1797x over previous
import functools

import jax
import jax.numpy as jnp
from jax import lax
from jax.experimental import pallas as pl
from jax.experimental.pallas import tpu as pltpu
from jax.experimental.pallas import tpu_sc as plsc

F32 = jnp.float32
_BN = 256
_WIN = 128
_NSC = 2
_NTILES = 16


def _tc_project(x_p, Wf, al, ar):
    Np, F = x_p.shape
    H, D = al.shape
    nb = Np // _BN
    al3 = al.reshape(H, 1, D)
    ar3 = ar.reshape(H, 1, D)

    def body(x_ref, w_ref, al_ref, ar_ref, f_ref, el_ref, er_ref):
        fb = jnp.dot(x_ref[...], w_ref[...], preferred_element_type=F32)
        f_ref[0] = fb
        el_ref[0] = jnp.sum(fb * al_ref[0], axis=1, keepdims=True)
        er_ref[0] = jnp.sum(fb * ar_ref[0], axis=1, keepdims=True)

    return pl.pallas_call(
        body,
        grid=(H, nb),
        in_specs=[
            pl.BlockSpec((_BN, F), lambda h, i: (i, 0)),
            pl.BlockSpec((F, D), lambda h, i: (0, h)),
            pl.BlockSpec((1, 1, D), lambda h, i: (h, 0, 0)),
            pl.BlockSpec((1, 1, D), lambda h, i: (h, 0, 0)),
        ],
        out_specs=[
            pl.BlockSpec((1, _BN, D), lambda h, i: (h, i, 0)),
            pl.BlockSpec((1, _BN, 1), lambda h, i: (h, i, 0)),
            pl.BlockSpec((1, _BN, 1), lambda h, i: (h, i, 0)),
        ],
        out_shape=[
            jax.ShapeDtypeStruct((H, Np, D), F32),
            jax.ShapeDtypeStruct((H, Np, 1), F32),
            jax.ShapeDtypeStruct((H, Np, 1), F32),
        ],
    )(x_p, Wf, al3, ar3)


def _sc_aggregate(src_p, dst_p, el_f, er_f, feat_f, Np, D, hps, edges_all):
    Ep = src_p.shape[0]
    nslab = _NSC * hps
    rpt = Np // _NTILES
    ept = Ep // _NTILES if edges_all else Ep // (_NTILES * _NSC)
    nwin = ept // _WIN
    zrows = jnp.zeros((rpt, D), F32)
    zden = jnp.zeros((hps * rpt,), F32)
    mesh = plsc.VectorSubcoreMesh(core_axis_name="c", subcore_axis_name="s")

    @functools.partial(
        pl.kernel,
        out_type=[
            jax.ShapeDtypeStruct((nslab, Np, D), F32),
            jax.ShapeDtypeStruct((nslab, Np), F32),
        ],
        mesh=mesh,
        compiler_params=pltpu.CompilerParams(needs_layout_passes=False),
        scratch_types=[
            pltpu.VMEM_SHARED((Np, D), F32),
            pltpu.VMEM_SHARED((hps * Np,), F32),
            pltpu.VMEM((_WIN,), jnp.int32),
            pltpu.VMEM((1, _WIN), jnp.int32),
            pltpu.VMEM((1, _WIN), jnp.int32),
            pltpu.VMEM((_WIN,), jnp.int32),
            pltpu.VMEM((_WIN,), jnp.int32),
            pltpu.VMEM((_WIN,), F32),
            pltpu.VMEM((_WIN,), F32),
            pltpu.VMEM((_WIN,), F32),
            pltpu.VMEM((_WIN, D), F32),
        ],
    )
    def k(src_h, dst_h, el_h, er_h, feat_h, zr_h, zd_h, out_h, den_h,
          out_sp, den_sp, src_v, dstr_v, dstd_v, srca_v, era_v,
          els_v, erd_v, ee_v, feat_v):
        c = lax.axis_index("c")
        s = lax.axis_index("s")
        pltpu.sync_copy(zr_h, out_sp.at[pl.ds(s * rpt, rpt)])
        pltpu.sync_copy(zd_h, den_sp.at[pl.ds(s * (hps * rpt), hps * rpt)])
        plsc.subcore_barrier()
        for p in range(hps):
            head = (c * hps + p) if edges_all else p
            hN = head * Np
            tile_base = (s * ept) if edges_all else (c * _NTILES + s) * ept

            @pl.loop(0, nwin)
            def _win(w):
                base = tile_base + w * _WIN
                pltpu.sync_copy(src_h.at[pl.ds(base, _WIN)], src_v)
                pltpu.sync_copy(dst_h.at[pl.ds(base, _WIN)], dstr_v.at[0])
                for kk in range(_WIN // 16):
                    sl = pl.ds(kk * 16, 16)
                    sv = src_v[sl]
                    dv = dstr_v[0, sl]
                    srca_v[sl] = sv + hN
                    era_v[sl] = dv + hN
                    dstd_v[0, sl] = dv + (p * Np)
                pltpu.sync_copy(el_h.at[srca_v], els_v)
                pltpu.sync_copy(er_h.at[era_v], erd_v)
                for kk in range(_WIN // 16):
                    sl = pl.ds(kk * 16, 16)
                    e = els_v[sl] + erd_v[sl]
                    e = jnp.where(e >= 0.0, e, 0.2 * e)
                    ee_v[sl] = jnp.exp(e)
                pltpu.sync_copy(ee_v, den_sp.at[dstd_v.at[0]], add=True)
                pltpu.sync_copy(feat_h.at[srca_v], feat_v)

                @pl.loop(0, _WIN)
                def _edge(ei):
                    eev = plsc.load_gather(
                        ee_v, [jnp.full((16,), ei, jnp.int32)])
                    for k2 in range(D // 16):
                        sl2 = pl.ds(k2 * 16, 16)
                        feat_v[ei, sl2] = feat_v[ei, sl2] * eev

                pltpu.sync_copy(feat_v, out_sp.at[dstr_v.at[0]], add=True)

            plsc.subcore_barrier()
            slab = c * hps + p
            pltpu.sync_copy(out_sp.at[pl.ds(s * rpt, rpt)],
                            out_h.at[slab, pl.ds(s * rpt, rpt)])
            if p < hps - 1:
                pltpu.sync_copy(zr_h, out_sp.at[pl.ds(s * rpt, rpt)])
                plsc.subcore_barrier()
        for p in range(hps):
            pltpu.sync_copy(den_sp.at[pl.ds(p * Np + s * rpt, rpt)],
                            den_h.at[c * hps + p, pl.ds(s * rpt, rpt)])

    return k(src_p, dst_p, el_f, er_f, feat_f, zrows, zden)


def _tc_norm_stats(out_slabs, den_slabs, b, n_real, sum_parts):
    S, Np, D = out_slabs.shape
    H = b.shape[0]
    HD = H * D
    nb = Np // _BN

    def body(o_ref, d_ref, b_ref, val_ref, st_ref):
        i = pl.program_id(0)
        if sum_parts:
            acc = o_ref[0]
            den = d_ref[0]
            for t in range(1, S):
                acc = acc + o_ref[t]
                den = den + d_ref[t]
            val = acc / (den + 1e-9) + b_ref[0][None, :]
        else:
            cols = []
            for hh in range(H):
                cols.append(o_ref[hh] / (d_ref[hh] + 1e-9)
                            + b_ref[hh][None, :])
            val = jnp.concatenate(cols, axis=1)
        rows = i * _BN + lax.broadcasted_iota(jnp.int32, (_BN, 1), 0)
        val = jnp.where(rows < n_real, val, 0.0)
        val_ref[...] = val

        @pl.when(i == 0)
        def _():
            st_ref[...] = jnp.zeros_like(st_ref)

        st_ref[0:1, :] += jnp.sum(val, axis=0, keepdims=True)
        st_ref[1:2, :] += jnp.sum(val * val, axis=0, keepdims=True)

    return pl.pallas_call(
        body,
        grid=(nb,),
        in_specs=[
            pl.BlockSpec((S, _BN, D), lambda i: (0, i, 0)),
            pl.BlockSpec((S, _BN, 1), lambda i: (0, i, 0)),
            pl.BlockSpec((H, D), lambda i: (0, 0)),
        ],
        out_specs=[
            pl.BlockSpec((_BN, HD), lambda i: (i, 0)),
            pl.BlockSpec((8, HD), lambda i: (0, 0)),
        ],
        out_shape=[
            jax.ShapeDtypeStruct((Np, HD), F32),
            jax.ShapeDtypeStruct((8, HD), F32),
        ],
    )(out_slabs, den_slabs, b)


def _tc_bn_relu(val, st, g, be, n_real):
    Np, HD = val.shape
    nb = Np // _BN

    def body(v_ref, st_ref, g_ref, be_ref, o_ref):
        i = pl.program_id(0)
        mean = st_ref[0:1, :] / n_real
        var = st_ref[1:2, :] / n_real - mean * mean
        inv = lax.rsqrt(var + 1e-5)
        hn = (v_ref[...] - mean) * inv * g_ref[...] + be_ref[...]
        hn = jnp.maximum(hn, 0.0)
        rows = i * _BN + lax.broadcasted_iota(jnp.int32, (_BN, 1), 0)
        o_ref[...] = jnp.where(rows < n_real, hn, 0.0)

    return pl.pallas_call(
        body,
        grid=(nb,),
        in_specs=[
            pl.BlockSpec((_BN, HD), lambda i: (i, 0)),
            pl.BlockSpec((8, HD), lambda i: (0, 0)),
            pl.BlockSpec((1, HD), lambda i: (0, 0)),
            pl.BlockSpec((1, HD), lambda i: (0, 0)),
        ],
        out_specs=pl.BlockSpec((_BN, HD), lambda i: (i, 0)),
        out_shape=jax.ShapeDtypeStruct((Np, HD), F32),
    )(val, st, g, be)


def _tc_head(h2, Wout, bout):
    Np, D = h2.shape
    C = Wout.shape[1]
    nb = Np // _BN

    def body(h_ref, w_ref, bo_ref, f_ref, o_ref):
        hb = h_ref[...]
        n2 = jnp.sum(hb * hb, axis=1, keepdims=True)
        nr = jnp.maximum(jnp.sqrt(n2), 1e-12)
        ft = hb / nr
        f_ref[...] = ft
        o_ref[...] = jnp.dot(ft, w_ref[...],
                             preferred_element_type=F32) + bo_ref[...]

    return pl.pallas_call(
        body,
        grid=(nb,),
        in_specs=[
            pl.BlockSpec((_BN, D), lambda i: (i, 0)),
            pl.BlockSpec((D, C), lambda i: (0, 0)),
            pl.BlockSpec((1, C), lambda i: (0, 0)),
        ],
        out_specs=[
            pl.BlockSpec((_BN, D), lambda i: (i, 0)),
            pl.BlockSpec((_BN, C), lambda i: (i, 0)),
        ],
        out_shape=[
            jax.ShapeDtypeStruct((Np, D), F32),
            jax.ShapeDtypeStruct((Np, C), F32),
        ],
    )(h2, Wout, bout)


def kernel(x, edge_index_0, edge_index_1, W0, al0, ar0, b0, g0, be0,
           W1, al1, ar1, b1, g1, be1, Wout, bout):
    N, F = x.shape
    H, D = al0.shape
    C = Wout.shape[1]
    E = edge_index_0.shape[1]

    Np = ((N + 64 + _BN - 1) // _BN) * _BN
    egran = _NSC * _NTILES * _WIN
    Ep = ((E + egran - 1) // egran) * egran
    pad_idx = N + (jnp.arange(Ep - E, dtype=jnp.int32) % 64)

    def pad_edges(ei):
        return (jnp.concatenate([ei[0], pad_idx]),
                jnp.concatenate([ei[1], pad_idx]))

    x_p = jnp.zeros((Np, F), F32).at[:N].set(x)
    src0, dst0 = pad_edges(edge_index_0)
    src1, dst1 = pad_edges(edge_index_1)

    feat0, el0, er0 = _tc_project(x_p, W0.reshape(F, H * D), al0, ar0)
    out0, den0 = _sc_aggregate(src0, dst0, el0.reshape(-1), er0.reshape(-1),
                               feat0.reshape(H * Np, D), Np, D,
                               hps=H // _NSC, edges_all=True)
    val0, st0 = _tc_norm_stats(out0, den0.reshape(H, Np, 1), b0, N,
                               sum_parts=False)
    h1 = _tc_bn_relu(val0, st0, g0.reshape(1, H * D), be0.reshape(1, H * D), N)

    feat1, el1, er1 = _tc_project(h1, W1.reshape(H * D, D), al1, ar1)
    out1, den1 = _sc_aggregate(src1, dst1, el1.reshape(-1), er1.reshape(-1),
                               feat1.reshape(Np, D), Np, D,
                               hps=1, edges_all=False)
    val1, st1 = _tc_norm_stats(out1, den1.reshape(_NSC, Np, 1), b1, N,
                               sum_parts=True)
    h2 = _tc_bn_relu(val1, st1, g1.reshape(1, D), be1.reshape(1, D), N)

    feat_out, logits = _tc_head(h2, Wout, bout.reshape(1, C))
    return (logits[:N], feat_out[:N])


# speedup vs baseline: 31.5600x; 1.7360x over previous
import functools

import jax
import jax.numpy as jnp
from jax import lax
from jax.experimental import pallas as pl
from jax.experimental.pallas import tpu as pltpu
from jax.experimental.pallas import tpu_sc as plsc

F32 = jnp.float32
_BN = 256
_WIN = 128
_NSC = 2
_NTILES = 16


def _tc_project(x_p, Wf, al, ar):
    Np, F = x_p.shape
    H, D = al.shape
    nb = Np // _BN
    al3 = al.reshape(H, 1, D)
    ar3 = ar.reshape(H, 1, D)

    def body(x_ref, w_ref, al_ref, ar_ref, f_ref, el_ref, er_ref):
        fb = jnp.dot(x_ref[...], w_ref[...], preferred_element_type=F32)
        f_ref[0] = fb
        el_ref[0] = jnp.sum(fb * al_ref[0], axis=1, keepdims=True)
        er_ref[0] = jnp.sum(fb * ar_ref[0], axis=1, keepdims=True)

    return pl.pallas_call(
        body,
        grid=(H, nb),
        in_specs=[
            pl.BlockSpec((_BN, F), lambda h, i: (i, 0)),
            pl.BlockSpec((F, D), lambda h, i: (0, h)),
            pl.BlockSpec((1, 1, D), lambda h, i: (h, 0, 0)),
            pl.BlockSpec((1, 1, D), lambda h, i: (h, 0, 0)),
        ],
        out_specs=[
            pl.BlockSpec((1, _BN, D), lambda h, i: (h, i, 0)),
            pl.BlockSpec((1, _BN, 1), lambda h, i: (h, i, 0)),
            pl.BlockSpec((1, _BN, 1), lambda h, i: (h, i, 0)),
        ],
        out_shape=[
            jax.ShapeDtypeStruct((H, Np, D), F32),
            jax.ShapeDtypeStruct((H, Np, 1), F32),
            jax.ShapeDtypeStruct((H, Np, 1), F32),
        ],
    )(x_p, Wf, al3, ar3)


def _sc_aggregate(src_p, dst_p, el_f, er_f, feat_f, Np, D, hps, edges_all):
    Ep = src_p.shape[0]
    nslab = _NSC * hps
    rpt = Np // _NTILES
    ept = Ep // _NTILES if edges_all else Ep // (_NTILES * _NSC)
    nwin = ept // _WIN
    assert nwin % 2 == 0
    zrows = jnp.zeros((rpt, D), F32)
    zden = jnp.zeros((hps * rpt,), F32)
    mesh = plsc.VectorSubcoreMesh(core_axis_name="c", subcore_axis_name="s")

    @functools.partial(
        pl.kernel,
        out_type=[
            jax.ShapeDtypeStruct((nslab, Np, D), F32),
            jax.ShapeDtypeStruct((nslab, Np), F32),
        ],
        mesh=mesh,
        compiler_params=pltpu.CompilerParams(needs_layout_passes=False),
        scratch_types=[
            pltpu.VMEM_SHARED((Np, D), F32),
            pltpu.VMEM_SHARED((hps * Np,), F32),
            pltpu.VMEM((2, _WIN), jnp.int32),
            pltpu.VMEM((2, _WIN), jnp.int32),
            pltpu.VMEM((2, _WIN), jnp.int32),
            pltpu.VMEM((2, _WIN), jnp.int32),
            pltpu.VMEM((2, _WIN), F32),
            pltpu.VMEM((2, _WIN), F32),
            pltpu.VMEM((2, _WIN), F32),
            pltpu.VMEM((2, _WIN, D), F32),
            pltpu.SemaphoreType.DMA,
            pltpu.SemaphoreType.DMA,
            pltpu.SemaphoreType.DMA,
            pltpu.SemaphoreType.DMA,
        ],
    )
    def k(src_h, dst_h, el_h, er_h, feat_h, zr_h, zd_h, out_h, den_h,
          out_sp, den_sp, srca_v, dstr_v, era_v, dstd_v,
          els_v, erd_v, ee_v, feat_v, gsem0, gsem1, ssem0, ssem1):
        c = lax.axis_index("c")
        s = lax.axis_index("s")
        gsem = (gsem0, gsem1)
        ssem = (ssem0, ssem1)
        pltpu.sync_copy(zr_h, out_sp.at[pl.ds(s * rpt, rpt)])
        pltpu.sync_copy(zd_h, den_sp.at[pl.ds(s * (hps * rpt), hps * rpt)])
        plsc.subcore_barrier()
        for p in range(hps):
            head = (c * hps + p) if edges_all else p
            hN = head * Np
            tile_base = (s * ept) if edges_all else (c * _NTILES + s) * ept

            def prefetch(wn, b):
                base = tile_base + wn * _WIN
                pltpu.sync_copy(src_h.at[pl.ds(base, _WIN)], srca_v.at[b])
                pltpu.sync_copy(dst_h.at[pl.ds(base, _WIN)], dstr_v.at[b])
                for kk in range(_WIN // 16):
                    sl = pl.ds(kk * 16, 16)
                    sv = srca_v[b, sl]
                    dv = dstr_v[b, sl]
                    srca_v[b, sl] = sv + hN
                    era_v[b, sl] = dv + hN
                    dstd_v[b, sl] = dv + (p * Np)
                pltpu.async_copy(el_h.at[srca_v.at[b]], els_v.at[b], gsem[b])
                pltpu.async_copy(er_h.at[era_v.at[b]], erd_v.at[b], gsem[b])
                pltpu.async_copy(feat_h.at[srca_v.at[b]], feat_v.at[b],
                                 gsem[b])

            def drain_gather(b):
                pltpu.make_async_copy(
                    el_h.at[pl.ds(0, _WIN)], els_v.at[b], gsem[b]).wait()
                pltpu.make_async_copy(
                    er_h.at[pl.ds(0, _WIN)], erd_v.at[b], gsem[b]).wait()
                pltpu.make_async_copy(
                    feat_h.at[pl.ds(0, _WIN)], feat_v.at[b], gsem[b]).wait()

            def drain_scatter(b):
                pltpu.make_async_copy(
                    ee_v.at[b], den_sp.at[pl.ds(0, _WIN)], ssem[b]).wait()
                pltpu.make_async_copy(
                    feat_v.at[b], out_sp.at[pl.ds(0, _WIN)], ssem[b]).wait()

            def compute(b):
                drain_gather(b)
                for kk in range(_WIN // 16):
                    sl = pl.ds(kk * 16, 16)
                    e = els_v[b, sl] + erd_v[b, sl]
                    e = jnp.where(e >= 0.0, e, 0.2 * e)
                    ee_v[b, sl] = jnp.exp(e)
                pltpu.async_copy(ee_v.at[b], den_sp.at[dstd_v.at[b]],
                                 ssem[b], add=True)

                @pl.loop(0, _WIN, unroll=4)
                def _edge(ei):
                    eev = plsc.load_gather(
                        ee_v.at[b], [jnp.full((16,), ei, jnp.int32)])
                    for k2 in range(D // 16):
                        sl2 = pl.ds(k2 * 16, 16)
                        feat_v[b, ei, sl2] = feat_v[b, ei, sl2] * eev

                pltpu.async_copy(feat_v.at[b], out_sp.at[dstr_v.at[b]],
                                 ssem[b], add=True)

            prefetch(0, 0)

            @pl.loop(0, nwin // 2)
            def _w2(i):
                for b in range(2):
                    w = 2 * i + b
                    b1 = 1 - b

                    @pl.when(w > 0)
                    def _():
                        drain_scatter(b1)

                    prefetch(jnp.minimum(w + 1, nwin - 1), b1)
                    compute(b)

            drain_scatter(1)
            drain_gather(0)
            plsc.subcore_barrier()
            slab = c * hps + p
            pltpu.sync_copy(out_sp.at[pl.ds(s * rpt, rpt)],
                            out_h.at[slab, pl.ds(s * rpt, rpt)])
            if p < hps - 1:
                pltpu.sync_copy(zr_h, out_sp.at[pl.ds(s * rpt, rpt)])
                plsc.subcore_barrier()
        for p in range(hps):
            pltpu.sync_copy(den_sp.at[pl.ds(p * Np + s * rpt, rpt)],
                            den_h.at[c * hps + p, pl.ds(s * rpt, rpt)])

    return k(src_p, dst_p, el_f, er_f, feat_f, zrows, zden)


def _tc_norm_stats(out_slabs, den_slabs, b, n_real, sum_parts):
    S, Np, D = out_slabs.shape
    H = b.shape[0]
    HD = H * D
    nb = Np // _BN

    def body(o_ref, d_ref, b_ref, val_ref, st_ref):
        i = pl.program_id(0)
        if sum_parts:
            acc = o_ref[0]
            den = d_ref[0]
            for t in range(1, S):
                acc = acc + o_ref[t]
                den = den + d_ref[t]
            val = acc / (den + 1e-9) + b_ref[0][None, :]
        else:
            cols = []
            for hh in range(H):
                cols.append(o_ref[hh] / (d_ref[hh] + 1e-9)
                            + b_ref[hh][None, :])
            val = jnp.concatenate(cols, axis=1)
        rows = i * _BN + lax.broadcasted_iota(jnp.int32, (_BN, 1), 0)
        val = jnp.where(rows < n_real, val, 0.0)
        val_ref[...] = val

        @pl.when(i == 0)
        def _():
            st_ref[...] = jnp.zeros_like(st_ref)

        st_ref[0:1, :] += jnp.sum(val, axis=0, keepdims=True)
        st_ref[1:2, :] += jnp.sum(val * val, axis=0, keepdims=True)

    return pl.pallas_call(
        body,
        grid=(nb,),
        in_specs=[
            pl.BlockSpec((S, _BN, D), lambda i: (0, i, 0)),
            pl.BlockSpec((S, _BN, 1), lambda i: (0, i, 0)),
            pl.BlockSpec((H, D), lambda i: (0, 0)),
        ],
        out_specs=[
            pl.BlockSpec((_BN, HD), lambda i: (i, 0)),
            pl.BlockSpec((8, HD), lambda i: (0, 0)),
        ],
        out_shape=[
            jax.ShapeDtypeStruct((Np, HD), F32),
            jax.ShapeDtypeStruct((8, HD), F32),
        ],
    )(out_slabs, den_slabs, b)


def _tc_bn_relu(val, st, g, be, n_real):
    Np, HD = val.shape
    nb = Np // _BN

    def body(v_ref, st_ref, g_ref, be_ref, o_ref):
        i = pl.program_id(0)
        mean = st_ref[0:1, :] / n_real
        var = st_ref[1:2, :] / n_real - mean * mean
        inv = lax.rsqrt(var + 1e-5)
        hn = (v_ref[...] - mean) * inv * g_ref[...] + be_ref[...]
        hn = jnp.maximum(hn, 0.0)
        rows = i * _BN + lax.broadcasted_iota(jnp.int32, (_BN, 1), 0)
        o_ref[...] = jnp.where(rows < n_real, hn, 0.0)

    return pl.pallas_call(
        body,
        grid=(nb,),
        in_specs=[
            pl.BlockSpec((_BN, HD), lambda i: (i, 0)),
            pl.BlockSpec((8, HD), lambda i: (0, 0)),
            pl.BlockSpec((1, HD), lambda i: (0, 0)),
            pl.BlockSpec((1, HD), lambda i: (0, 0)),
        ],
        out_specs=pl.BlockSpec((_BN, HD), lambda i: (i, 0)),
        out_shape=jax.ShapeDtypeStruct((Np, HD), F32),
    )(val, st, g, be)


def _tc_head(h2, Wout, bout):
    Np, D = h2.shape
    C = Wout.shape[1]
    nb = Np // _BN

    def body(h_ref, w_ref, bo_ref, f_ref, o_ref):
        hb = h_ref[...]
        n2 = jnp.sum(hb * hb, axis=1, keepdims=True)
        nr = jnp.maximum(jnp.sqrt(n2), 1e-12)
        ft = hb / nr
        f_ref[...] = ft
        o_ref[...] = jnp.dot(ft, w_ref[...],
                             preferred_element_type=F32) + bo_ref[...]

    return pl.pallas_call(
        body,
        grid=(nb,),
        in_specs=[
            pl.BlockSpec((_BN, D), lambda i: (i, 0)),
            pl.BlockSpec((D, C), lambda i: (0, 0)),
            pl.BlockSpec((1, C), lambda i: (0, 0)),
        ],
        out_specs=[
            pl.BlockSpec((_BN, D), lambda i: (i, 0)),
            pl.BlockSpec((_BN, C), lambda i: (i, 0)),
        ],
        out_shape=[
            jax.ShapeDtypeStruct((Np, D), F32),
            jax.ShapeDtypeStruct((Np, C), F32),
        ],
    )(h2, Wout, bout)


def kernel(x, edge_index_0, edge_index_1, W0, al0, ar0, b0, g0, be0,
           W1, al1, ar1, b1, g1, be1, Wout, bout):
    N, F = x.shape
    H, D = al0.shape
    C = Wout.shape[1]
    E = edge_index_0.shape[1]

    Np = ((N + 64 + _BN - 1) // _BN) * _BN
    egran = 2 * _NSC * _NTILES * _WIN
    Ep = ((E + egran - 1) // egran) * egran
    pad_idx = N + (jnp.arange(Ep - E, dtype=jnp.int32) % 64)

    def pad_edges(ei):
        return (jnp.concatenate([ei[0], pad_idx]),
                jnp.concatenate([ei[1], pad_idx]))

    x_p = jnp.zeros((Np, F), F32).at[:N].set(x)
    src0, dst0 = pad_edges(edge_index_0)
    src1, dst1 = pad_edges(edge_index_1)

    feat0, el0, er0 = _tc_project(x_p, W0.reshape(F, H * D), al0, ar0)
    out0, den0 = _sc_aggregate(src0, dst0, el0.reshape(-1), er0.reshape(-1),
                               feat0.reshape(H * Np, D), Np, D,
                               hps=H // _NSC, edges_all=True)
    val0, st0 = _tc_norm_stats(out0, den0.reshape(H, Np, 1), b0, N,
                               sum_parts=False)
    h1 = _tc_bn_relu(val0, st0, g0.reshape(1, H * D), be0.reshape(1, H * D), N)

    feat1, el1, er1 = _tc_project(h1, W1.reshape(H * D, D), al1, ar1)
    out1, den1 = _sc_aggregate(src1, dst1, el1.reshape(-1), er1.reshape(-1),
                               feat1.reshape(Np, D), Np, D,
                               hps=1, edges_all=False)
    val1, st1 = _tc_norm_stats(out1, den1.reshape(_NSC, Np, 1), b1, N,
                               sum_parts=True)
    h2 = _tc_bn_relu(val1, st1, g1.reshape(1, D), be1.reshape(1, D), N)

    feat_out, logits = _tc_head(h2, Wout, bout.reshape(1, C))
    return (logits[:N], feat_out[:N])


# speedup vs baseline: 39.1378x; 1.2401x over previous
import functools

import jax
import jax.numpy as jnp
from jax import lax
from jax.experimental import pallas as pl
from jax.experimental.pallas import tpu as pltpu
from jax.experimental.pallas import tpu_sc as plsc

F32 = jnp.float32
_BN = 256
_WIN = 128
_NSC = 2
_NTILES = 16


def _tc_project(x_p, Wf, al, ar):
    Np, F = x_p.shape
    H, D = al.shape
    nb = Np // _BN
    al3 = al.reshape(H, 1, D)
    ar3 = ar.reshape(H, 1, D)

    def body(x_ref, w_ref, al_ref, ar_ref, f_ref, el_ref, er_ref):
        fb = jnp.dot(x_ref[...], w_ref[...], preferred_element_type=F32)
        f_ref[0] = fb
        el_ref[0] = jnp.sum(fb * al_ref[0], axis=1, keepdims=True)
        er_ref[0] = jnp.sum(fb * ar_ref[0], axis=1, keepdims=True)

    return pl.pallas_call(
        body,
        grid=(H, nb),
        in_specs=[
            pl.BlockSpec((_BN, F), lambda h, i: (i, 0)),
            pl.BlockSpec((F, D), lambda h, i: (0, h)),
            pl.BlockSpec((1, 1, D), lambda h, i: (h, 0, 0)),
            pl.BlockSpec((1, 1, D), lambda h, i: (h, 0, 0)),
        ],
        out_specs=[
            pl.BlockSpec((1, _BN, D), lambda h, i: (h, i, 0)),
            pl.BlockSpec((1, _BN, 1), lambda h, i: (h, i, 0)),
            pl.BlockSpec((1, _BN, 1), lambda h, i: (h, i, 0)),
        ],
        out_shape=[
            jax.ShapeDtypeStruct((H, Np, D), F32),
            jax.ShapeDtypeStruct((H, Np, 1), F32),
            jax.ShapeDtypeStruct((H, Np, 1), F32),
        ],
    )(x_p, Wf, al3, ar3)


def _sc_aggregate(src_p, dst_p, el_f, er_f, feat_f, Np, D, hps, edges_all):
    Ep = src_p.shape[0]
    nslab = _NSC * hps
    rpt = Np // _NTILES
    ept = Ep // _NTILES if edges_all else Ep // (_NTILES * _NSC)
    nwin = ept // _WIN
    assert nwin % 2 == 0
    ch = 8
    ce = ch * _WIN
    assert nwin % ch == 0
    zrows = jnp.zeros((rpt, D), F32)
    zden = jnp.zeros((hps * rpt,), F32)
    mesh = plsc.VectorSubcoreMesh(core_axis_name="c", subcore_axis_name="s")

    @functools.partial(
        pl.kernel,
        out_type=[
            jax.ShapeDtypeStruct((nslab, Np, D), F32),
            jax.ShapeDtypeStruct((nslab, Np), F32),
        ],
        mesh=mesh,
        compiler_params=pltpu.CompilerParams(needs_layout_passes=False),
        scratch_types=[
            pltpu.VMEM_SHARED((Np, D), F32),
            pltpu.VMEM_SHARED((hps * Np,), F32),
            pltpu.VMEM((2, _WIN), jnp.int32),
            pltpu.VMEM((2, _WIN), jnp.int32),
            pltpu.VMEM((2, _WIN), jnp.int32),
            pltpu.VMEM((2, _WIN), jnp.int32),
            pltpu.VMEM((2, _WIN), F32),
            pltpu.VMEM((2, _WIN), F32),
            pltpu.VMEM((2, _WIN), F32),
            pltpu.VMEM((2, _WIN, D), F32),
            pltpu.VMEM((ce,), jnp.int32),
            pltpu.VMEM((ce,), jnp.int32),
            pltpu.SemaphoreType.DMA,
            pltpu.SemaphoreType.DMA,
            pltpu.SemaphoreType.DMA,
            pltpu.SemaphoreType.DMA,
        ],
    )
    def k(src_h, dst_h, el_h, er_h, feat_h, zr_h, zd_h, out_h, den_h,
          out_sp, den_sp, srca_v, dstr_v, era_v, dstd_v,
          els_v, erd_v, ee_v, feat_v, srcall_v, dstall_v,
          gsem0, gsem1, ssem0, ssem1):
        c = lax.axis_index("c")
        s = lax.axis_index("s")
        gsem = (gsem0, gsem1)
        ssem = (ssem0, ssem1)
        tile_base = (s * ept) if edges_all else (c * _NTILES + s) * ept
        pltpu.sync_copy(zr_h, out_sp.at[pl.ds(s * rpt, rpt)])
        pltpu.sync_copy(zd_h, den_sp.at[pl.ds(s * (hps * rpt), hps * rpt)])
        plsc.subcore_barrier()
        for p in range(hps):
            head = (c * hps + p) if edges_all else p
            hN = head * Np

            def prefetch(wn, b):
                @pl.when(lax.rem(wn, ch) == 0)
                def _():
                    base = tile_base + wn * _WIN
                    pltpu.sync_copy(src_h.at[pl.ds(base, ce)], srcall_v)
                    pltpu.sync_copy(dst_h.at[pl.ds(base, ce)], dstall_v)

                wofs = lax.rem(wn, ch) * _WIN
                for kk in range(_WIN // 16):
                    sl = pl.ds(kk * 16, 16)
                    lsl = pl.ds(wofs + kk * 16, 16)
                    sv = srcall_v[lsl]
                    dv = dstall_v[lsl]
                    srca_v[b, sl] = sv + hN
                    dstr_v[b, sl] = dv
                    era_v[b, sl] = dv + hN
                    dstd_v[b, sl] = dv + (p * Np)
                pltpu.async_copy(el_h.at[srca_v.at[b]], els_v.at[b], gsem[b])
                pltpu.async_copy(er_h.at[era_v.at[b]], erd_v.at[b], gsem[b])
                pltpu.async_copy(feat_h.at[srca_v.at[b]], feat_v.at[b],
                                 gsem[b])

            def drain_gather(b):
                pltpu.make_async_copy(
                    el_h.at[pl.ds(0, _WIN)], els_v.at[b], gsem[b]).wait()
                pltpu.make_async_copy(
                    er_h.at[pl.ds(0, _WIN)], erd_v.at[b], gsem[b]).wait()
                pltpu.make_async_copy(
                    feat_h.at[pl.ds(0, _WIN)], feat_v.at[b], gsem[b]).wait()

            def drain_scatter(b):
                pltpu.make_async_copy(
                    ee_v.at[b], den_sp.at[pl.ds(0, _WIN)], ssem[b]).wait()
                pltpu.make_async_copy(
                    feat_v.at[b], out_sp.at[pl.ds(0, _WIN)], ssem[b]).wait()

            def compute(b):
                drain_gather(b)
                for kk in range(_WIN // 16):
                    sl = pl.ds(kk * 16, 16)
                    e = els_v[b, sl] + erd_v[b, sl]
                    e = jnp.where(e >= 0.0, e, 0.2 * e)
                    ee_v[b, sl] = jnp.exp(e)
                pltpu.async_copy(ee_v.at[b], den_sp.at[dstd_v.at[b]],
                                 ssem[b], add=True)

                @pl.loop(0, _WIN, unroll=4)
                def _edge(ei):
                    eev = plsc.load_gather(
                        ee_v.at[b], [jnp.full((16,), ei, jnp.int32)])
                    for k2 in range(D // 16):
                        sl2 = pl.ds(k2 * 16, 16)
                        feat_v[b, ei, sl2] = feat_v[b, ei, sl2] * eev

                pltpu.async_copy(feat_v.at[b], out_sp.at[dstr_v.at[b]],
                                 ssem[b], add=True)

            prefetch(0, 0)

            @pl.loop(0, nwin // 2)
            def _w2(i):
                for b in range(2):
                    w = 2 * i + b
                    b1 = 1 - b

                    @pl.when(w > 0)
                    def _():
                        drain_scatter(b1)

                    prefetch(jnp.minimum(w + 1, nwin - 1), b1)
                    compute(b)

            drain_scatter(1)
            drain_gather(0)
            plsc.subcore_barrier()
            slab = c * hps + p
            pltpu.sync_copy(out_sp.at[pl.ds(s * rpt, rpt)],
                            out_h.at[slab, pl.ds(s * rpt, rpt)])
            if p < hps - 1:
                pltpu.sync_copy(zr_h, out_sp.at[pl.ds(s * rpt, rpt)])
                plsc.subcore_barrier()
        for p in range(hps):
            pltpu.sync_copy(den_sp.at[pl.ds(p * Np + s * rpt, rpt)],
                            den_h.at[c * hps + p, pl.ds(s * rpt, rpt)])

    return k(src_p, dst_p, el_f, er_f, feat_f, zrows, zden)


def _tc_norm_stats(out_slabs, den_slabs, b, n_real, sum_parts):
    S, Np, D = out_slabs.shape
    H = b.shape[0]
    HD = H * D
    nb = Np // _BN

    def body(o_ref, d_ref, b_ref, val_ref, st_ref):
        i = pl.program_id(0)
        if sum_parts:
            acc = o_ref[0]
            den = d_ref[0]
            for t in range(1, S):
                acc = acc + o_ref[t]
                den = den + d_ref[t]
            val = acc / (den + 1e-9) + b_ref[0][None, :]
        else:
            cols = []
            for hh in range(H):
                cols.append(o_ref[hh] / (d_ref[hh] + 1e-9)
                            + b_ref[hh][None, :])
            val = jnp.concatenate(cols, axis=1)
        rows = i * _BN + lax.broadcasted_iota(jnp.int32, (_BN, 1), 0)
        val = jnp.where(rows < n_real, val, 0.0)
        val_ref[...] = val

        @pl.when(i == 0)
        def _():
            st_ref[...] = jnp.zeros_like(st_ref)

        st_ref[0:1, :] += jnp.sum(val, axis=0, keepdims=True)
        st_ref[1:2, :] += jnp.sum(val * val, axis=0, keepdims=True)

    return pl.pallas_call(
        body,
        grid=(nb,),
        in_specs=[
            pl.BlockSpec((S, _BN, D), lambda i: (0, i, 0)),
            pl.BlockSpec((S, _BN, 1), lambda i: (0, i, 0)),
            pl.BlockSpec((H, D), lambda i: (0, 0)),
        ],
        out_specs=[
            pl.BlockSpec((_BN, HD), lambda i: (i, 0)),
            pl.BlockSpec((8, HD), lambda i: (0, 0)),
        ],
        out_shape=[
            jax.ShapeDtypeStruct((Np, HD), F32),
            jax.ShapeDtypeStruct((8, HD), F32),
        ],
    )(out_slabs, den_slabs, b)


def _tc_bn_relu(val, st, g, be, n_real):
    Np, HD = val.shape
    nb = Np // _BN

    def body(v_ref, st_ref, g_ref, be_ref, o_ref):
        i = pl.program_id(0)
        mean = st_ref[0:1, :] / n_real
        var = st_ref[1:2, :] / n_real - mean * mean
        inv = lax.rsqrt(var + 1e-5)
        hn = (v_ref[...] - mean) * inv * g_ref[...] + be_ref[...]
        hn = jnp.maximum(hn, 0.0)
        rows = i * _BN + lax.broadcasted_iota(jnp.int32, (_BN, 1), 0)
        o_ref[...] = jnp.where(rows < n_real, hn, 0.0)

    return pl.pallas_call(
        body,
        grid=(nb,),
        in_specs=[
            pl.BlockSpec((_BN, HD), lambda i: (i, 0)),
            pl.BlockSpec((8, HD), lambda i: (0, 0)),
            pl.BlockSpec((1, HD), lambda i: (0, 0)),
            pl.BlockSpec((1, HD), lambda i: (0, 0)),
        ],
        out_specs=pl.BlockSpec((_BN, HD), lambda i: (i, 0)),
        out_shape=jax.ShapeDtypeStruct((Np, HD), F32),
    )(val, st, g, be)


def _tc_head(h2, Wout, bout):
    Np, D = h2.shape
    C = Wout.shape[1]
    nb = Np // _BN

    def body(h_ref, w_ref, bo_ref, f_ref, o_ref):
        hb = h_ref[...]
        n2 = jnp.sum(hb * hb, axis=1, keepdims=True)
        nr = jnp.maximum(jnp.sqrt(n2), 1e-12)
        ft = hb / nr
        f_ref[...] = ft
        o_ref[...] = jnp.dot(ft, w_ref[...],
                             preferred_element_type=F32) + bo_ref[...]

    return pl.pallas_call(
        body,
        grid=(nb,),
        in_specs=[
            pl.BlockSpec((_BN, D), lambda i: (i, 0)),
            pl.BlockSpec((D, C), lambda i: (0, 0)),
            pl.BlockSpec((1, C), lambda i: (0, 0)),
        ],
        out_specs=[
            pl.BlockSpec((_BN, D), lambda i: (i, 0)),
            pl.BlockSpec((_BN, C), lambda i: (i, 0)),
        ],
        out_shape=[
            jax.ShapeDtypeStruct((Np, D), F32),
            jax.ShapeDtypeStruct((Np, C), F32),
        ],
    )(h2, Wout, bout)


def kernel(x, edge_index_0, edge_index_1, W0, al0, ar0, b0, g0, be0,
           W1, al1, ar1, b1, g1, be1, Wout, bout):
    N, F = x.shape
    H, D = al0.shape
    C = Wout.shape[1]
    E = edge_index_0.shape[1]

    Np = ((N + 64 + _BN - 1) // _BN) * _BN
    egran = 8 * _NSC * _NTILES * _WIN
    Ep = ((E + egran - 1) // egran) * egran
    pad_idx = N + (jnp.arange(Ep - E, dtype=jnp.int32) % 64)

    def pad_edges(ei):
        return (jnp.concatenate([ei[0], pad_idx]),
                jnp.concatenate([ei[1], pad_idx]))

    x_p = jnp.zeros((Np, F), F32).at[:N].set(x)
    src0, dst0 = pad_edges(edge_index_0)
    src1, dst1 = pad_edges(edge_index_1)

    feat0, el0, er0 = _tc_project(x_p, W0.reshape(F, H * D), al0, ar0)
    out0, den0 = _sc_aggregate(src0, dst0, el0.reshape(-1), er0.reshape(-1),
                               feat0.reshape(H * Np, D), Np, D,
                               hps=H // _NSC, edges_all=True)
    val0, st0 = _tc_norm_stats(out0, den0.reshape(H, Np, 1), b0, N,
                               sum_parts=False)
    h1 = _tc_bn_relu(val0, st0, g0.reshape(1, H * D), be0.reshape(1, H * D), N)

    feat1, el1, er1 = _tc_project(h1, W1.reshape(H * D, D), al1, ar1)
    out1, den1 = _sc_aggregate(src1, dst1, el1.reshape(-1), er1.reshape(-1),
                               feat1.reshape(Np, D), Np, D,
                               hps=1, edges_all=False)
    val1, st1 = _tc_norm_stats(out1, den1.reshape(_NSC, Np, 1), b1, N,
                               sum_parts=True)
    h2 = _tc_bn_relu(val1, st1, g1.reshape(1, D), be1.reshape(1, D), N)

    feat_out, logits = _tc_head(h2, Wout, bout.reshape(1, C))
    return (logits[:N], feat_out[:N])


# speedup vs baseline: 40.9272x; 1.0457x over previous
import functools

import jax
import jax.numpy as jnp
from jax import lax
from jax.experimental import pallas as pl
from jax.experimental.pallas import tpu as pltpu
from jax.experimental.pallas import tpu_sc as plsc

F32 = jnp.float32
_BN = 256
_WIN = 128
_NSC = 2
_NTILES = 16


def _tc_project(x_p, Wf, al, ar):
    Np, F = x_p.shape
    H, D = al.shape
    nb = Np // _BN
    al3 = al.reshape(H, 1, D)
    ar3 = ar.reshape(H, 1, D)

    def body(x_ref, w_ref, al_ref, ar_ref, f_ref, el_ref, er_ref):
        fb = jnp.dot(x_ref[...], w_ref[...], preferred_element_type=F32)
        f_ref[0] = fb
        el_ref[0] = jnp.sum(fb * al_ref[0], axis=1, keepdims=True)
        er_ref[0] = jnp.sum(fb * ar_ref[0], axis=1, keepdims=True)

    return pl.pallas_call(
        body,
        grid=(H, nb),
        in_specs=[
            pl.BlockSpec((_BN, F), lambda h, i: (i, 0)),
            pl.BlockSpec((F, D), lambda h, i: (0, h)),
            pl.BlockSpec((1, 1, D), lambda h, i: (h, 0, 0)),
            pl.BlockSpec((1, 1, D), lambda h, i: (h, 0, 0)),
        ],
        out_specs=[
            pl.BlockSpec((1, _BN, D), lambda h, i: (h, i, 0)),
            pl.BlockSpec((1, _BN, 1), lambda h, i: (h, i, 0)),
            pl.BlockSpec((1, _BN, 1), lambda h, i: (h, i, 0)),
        ],
        out_shape=[
            jax.ShapeDtypeStruct((H, Np, D), F32),
            jax.ShapeDtypeStruct((H, Np, 1), F32),
            jax.ShapeDtypeStruct((H, Np, 1), F32),
        ],
    )(x_p, Wf, al3, ar3)


def _sc_aggregate(src_p, dst_p, el_f, er_f, feat_f, Np, D, hps, edges_all):
    Ep = src_p.shape[0]
    nslab = _NSC * hps
    rpt = Np // _NTILES
    ept = Ep // _NTILES if edges_all else Ep // (_NTILES * _NSC)
    nwin = ept // _WIN
    assert nwin % 2 == 0
    ch = 8
    ce = ch * _WIN
    assert nwin % ch == 0
    zrows = jnp.zeros((rpt, D), F32)
    zden = jnp.zeros((hps * rpt,), F32)
    mesh = plsc.VectorSubcoreMesh(core_axis_name="c", subcore_axis_name="s")

    @functools.partial(
        pl.kernel,
        out_type=[
            jax.ShapeDtypeStruct((nslab, Np, D), F32),
            jax.ShapeDtypeStruct((nslab, Np), F32),
        ],
        mesh=mesh,
        compiler_params=pltpu.CompilerParams(needs_layout_passes=False),
        scratch_types=[
            pltpu.VMEM_SHARED((Np, D), F32),
            pltpu.VMEM_SHARED((hps * Np,), F32),
            pltpu.VMEM((2, _WIN), jnp.int32),
            pltpu.VMEM((2, _WIN), jnp.int32),
            pltpu.VMEM((2, _WIN), jnp.int32),
            pltpu.VMEM((2, _WIN), jnp.int32),
            pltpu.VMEM((2, _WIN), F32),
            pltpu.VMEM((2, _WIN), F32),
            pltpu.VMEM((2, _WIN), F32),
            pltpu.VMEM((2, _WIN, D), F32),
            pltpu.VMEM((ce,), jnp.int32),
            pltpu.VMEM((ce,), jnp.int32),
            pltpu.SemaphoreType.DMA,
            pltpu.SemaphoreType.DMA,
            pltpu.SemaphoreType.DMA,
            pltpu.SemaphoreType.DMA,
        ],
    )
    def k(src_h, dst_h, el_h, er_h, feat_h, zr_h, zd_h, out_h, den_h,
          out_sp, den_sp, srca_v, dstr_v, era_v, dstd_v,
          els_v, erd_v, ee_v, feat_v, srcall_v, dstall_v,
          gsem0, gsem1, ssem0, ssem1):
        c = lax.axis_index("c")
        s = lax.axis_index("s")
        gsem = (gsem0, gsem1)
        ssem = (ssem0, ssem1)
        tile_base = (s * ept) if edges_all else (c * _NTILES + s) * ept
        pltpu.sync_copy(zr_h, out_sp.at[pl.ds(s * rpt, rpt)])
        pltpu.sync_copy(zd_h, den_sp.at[pl.ds(s * (hps * rpt), hps * rpt)])
        plsc.subcore_barrier()
        for p in range(hps):
            head = (c * hps + p) if edges_all else p
            hN = head * Np

            def prefetch(wn, b):
                @pl.when(lax.rem(wn, ch) == 0)
                def _():
                    base = tile_base + wn * _WIN
                    pltpu.sync_copy(src_h.at[pl.ds(base, ce)], srcall_v)
                    pltpu.sync_copy(dst_h.at[pl.ds(base, ce)], dstall_v)

                wofs = lax.rem(wn, ch) * _WIN
                for kk in range(_WIN // 16):
                    sl = pl.ds(kk * 16, 16)
                    lsl = pl.ds(wofs + kk * 16, 16)
                    sv = srcall_v[lsl]
                    dv = dstall_v[lsl]
                    srca_v[b, sl] = sv + hN
                    dstr_v[b, sl] = dv
                    era_v[b, sl] = dv + hN
                    dstd_v[b, sl] = dv + (p * Np)
                pltpu.async_copy(el_h.at[srca_v.at[b]], els_v.at[b], gsem[b])
                pltpu.async_copy(er_h.at[era_v.at[b]], erd_v.at[b], gsem[b])
                pltpu.async_copy(feat_h.at[srca_v.at[b]], feat_v.at[b],
                                 gsem[b])

            def drain_gather(b):
                pltpu.make_async_copy(
                    el_h.at[pl.ds(0, _WIN)], els_v.at[b], gsem[b]).wait()
                pltpu.make_async_copy(
                    er_h.at[pl.ds(0, _WIN)], erd_v.at[b], gsem[b]).wait()
                pltpu.make_async_copy(
                    feat_h.at[pl.ds(0, _WIN)], feat_v.at[b], gsem[b]).wait()

            def drain_scatter(b):
                pltpu.make_async_copy(
                    ee_v.at[b], den_sp.at[pl.ds(0, _WIN)], ssem[b]).wait()
                pltpu.make_async_copy(
                    feat_v.at[b], out_sp.at[pl.ds(0, _WIN)], ssem[b]).wait()

            def compute(b):
                drain_gather(b)
                for kk in range(_WIN // 16):
                    sl = pl.ds(kk * 16, 16)
                    e = els_v[b, sl] + erd_v[b, sl]
                    e = jnp.where(e >= 0.0, e, 0.2 * e)
                    ee_v[b, sl] = jnp.exp(e)
                pltpu.async_copy(ee_v.at[b], den_sp.at[dstd_v.at[b]],
                                 ssem[b], add=True)

                @pl.loop(0, _WIN, unroll=4)
                def _edge(ei):
                    eev = plsc.load_gather(
                        ee_v.at[b], [jnp.full((16,), ei, jnp.int32)])
                    for k2 in range(D // 16):
                        sl2 = pl.ds(k2 * 16, 16)
                        feat_v[b, ei, sl2] = feat_v[b, ei, sl2] * eev

                pltpu.async_copy(feat_v.at[b], out_sp.at[dstr_v.at[b]],
                                 ssem[b], add=True)

            prefetch(0, 0)

            @pl.loop(0, nwin // 2)
            def _w2(i):
                for b in range(2):
                    w = 2 * i + b
                    b1 = 1 - b

                    @pl.when(w > 0)
                    def _():
                        drain_scatter(b1)

                    prefetch(jnp.minimum(w + 1, nwin - 1), b1)
                    compute(b)

            drain_scatter(1)
            drain_gather(0)
            plsc.subcore_barrier()
            slab = c * hps + p
            pltpu.sync_copy(out_sp.at[pl.ds(s * rpt, rpt)],
                            out_h.at[slab, pl.ds(s * rpt, rpt)])
            if p < hps - 1:
                pltpu.sync_copy(zr_h, out_sp.at[pl.ds(s * rpt, rpt)])
                plsc.subcore_barrier()
        for p in range(hps):
            pltpu.sync_copy(den_sp.at[pl.ds(p * Np + s * rpt, rpt)],
                            den_h.at[c * hps + p, pl.ds(s * rpt, rpt)])

    return k(src_p, dst_p, el_f, er_f, feat_f, zrows, zden)


def _tc_norm_stats(out_slabs, den_slabs, b, n_real, sum_parts):
    S, Np, D = out_slabs.shape
    H = b.shape[0]
    HD = H * D
    nb = Np // _BN

    def body(o_ref, d_ref, b_ref, val_ref, st_ref):
        i = pl.program_id(0)
        if sum_parts:
            acc = o_ref[0]
            den = d_ref[0]
            for t in range(1, S):
                acc = acc + o_ref[t]
                den = den + d_ref[t]
            val = acc / (den + 1e-9) + b_ref[0][None, :]
        else:
            cols = []
            for hh in range(H):
                cols.append(o_ref[hh] / (d_ref[hh] + 1e-9)
                            + b_ref[hh][None, :])
            val = jnp.concatenate(cols, axis=1)
        rows = i * _BN + lax.broadcasted_iota(jnp.int32, (_BN, 1), 0)
        val = jnp.where(rows < n_real, val, 0.0)
        val_ref[...] = val

        @pl.when(i == 0)
        def _():
            st_ref[...] = jnp.zeros_like(st_ref)

        st_ref[0:1, :] += jnp.sum(val, axis=0, keepdims=True)
        st_ref[1:2, :] += jnp.sum(val * val, axis=0, keepdims=True)

    return pl.pallas_call(
        body,
        grid=(nb,),
        in_specs=[
            pl.BlockSpec((S, _BN, D), lambda i: (0, i, 0)),
            pl.BlockSpec((S, _BN, 1), lambda i: (0, i, 0)),
            pl.BlockSpec((H, D), lambda i: (0, 0)),
        ],
        out_specs=[
            pl.BlockSpec((_BN, HD), lambda i: (i, 0)),
            pl.BlockSpec((8, HD), lambda i: (0, 0)),
        ],
        out_shape=[
            jax.ShapeDtypeStruct((Np, HD), F32),
            jax.ShapeDtypeStruct((8, HD), F32),
        ],
    )(out_slabs, den_slabs, b)


def _bn_block(v_ref, st_ref, g_ref, be_ref, i, n_real):
    mean = st_ref[0:1, :] / n_real
    var = st_ref[1:2, :] / n_real - mean * mean
    inv = lax.rsqrt(var + 1e-5)
    hn = (v_ref[...] - mean) * inv * g_ref[...] + be_ref[...]
    hn = jnp.maximum(hn, 0.0)
    rows = i * _BN + lax.broadcasted_iota(jnp.int32, (_BN, 1), 0)
    return jnp.where(rows < n_real, hn, 0.0)


def _tc_bn_project(val, st, g, be, n_real, Wf, al, ar):
    Np, HDin = val.shape
    H, D = al.shape
    nb = Np // _BN
    al3 = al.reshape(H, 1, D)
    ar3 = ar.reshape(H, 1, D)

    def body(v_ref, st_ref, g_ref, be_ref, w_ref, al_ref, ar_ref,
             f_ref, el_ref, er_ref):
        i = pl.program_id(1)
        hb = _bn_block(v_ref, st_ref, g_ref, be_ref, i, n_real)
        fb = jnp.dot(hb, w_ref[...], preferred_element_type=F32)
        f_ref[0] = fb
        el_ref[0] = jnp.sum(fb * al_ref[0], axis=1, keepdims=True)
        er_ref[0] = jnp.sum(fb * ar_ref[0], axis=1, keepdims=True)

    return pl.pallas_call(
        body,
        grid=(H, nb),
        in_specs=[
            pl.BlockSpec((_BN, HDin), lambda h, i: (i, 0)),
            pl.BlockSpec((8, HDin), lambda h, i: (0, 0)),
            pl.BlockSpec((1, HDin), lambda h, i: (0, 0)),
            pl.BlockSpec((1, HDin), lambda h, i: (0, 0)),
            pl.BlockSpec((HDin, D), lambda h, i: (0, h)),
            pl.BlockSpec((1, 1, D), lambda h, i: (h, 0, 0)),
            pl.BlockSpec((1, 1, D), lambda h, i: (h, 0, 0)),
        ],
        out_specs=[
            pl.BlockSpec((1, _BN, D), lambda h, i: (h, i, 0)),
            pl.BlockSpec((1, _BN, 1), lambda h, i: (h, i, 0)),
            pl.BlockSpec((1, _BN, 1), lambda h, i: (h, i, 0)),
        ],
        out_shape=[
            jax.ShapeDtypeStruct((H, Np, D), F32),
            jax.ShapeDtypeStruct((H, Np, 1), F32),
            jax.ShapeDtypeStruct((H, Np, 1), F32),
        ],
    )(val, st, g, be, Wf, al3, ar3)


def _tc_bn_head(val, st, g, be, n_real, Wout, bout):
    Np, D = val.shape
    C = Wout.shape[1]
    nb = Np // _BN

    def body(v_ref, st_ref, g_ref, be_ref, w_ref, bo_ref, f_ref, o_ref):
        i = pl.program_id(0)
        hb = _bn_block(v_ref, st_ref, g_ref, be_ref, i, n_real)
        n2 = jnp.sum(hb * hb, axis=1, keepdims=True)
        nr = jnp.maximum(jnp.sqrt(n2), 1e-12)
        ft = hb / nr
        f_ref[...] = ft
        o_ref[...] = jnp.dot(ft, w_ref[...],
                             preferred_element_type=F32) + bo_ref[...]

    return pl.pallas_call(
        body,
        grid=(nb,),
        in_specs=[
            pl.BlockSpec((_BN, D), lambda i: (i, 0)),
            pl.BlockSpec((8, D), lambda i: (0, 0)),
            pl.BlockSpec((1, D), lambda i: (0, 0)),
            pl.BlockSpec((1, D), lambda i: (0, 0)),
            pl.BlockSpec((D, C), lambda i: (0, 0)),
            pl.BlockSpec((1, C), lambda i: (0, 0)),
        ],
        out_specs=[
            pl.BlockSpec((_BN, D), lambda i: (i, 0)),
            pl.BlockSpec((_BN, C), lambda i: (i, 0)),
        ],
        out_shape=[
            jax.ShapeDtypeStruct((Np, D), F32),
            jax.ShapeDtypeStruct((Np, C), F32),
        ],
    )(val, st, g, be, Wout, bout)


def kernel(x, edge_index_0, edge_index_1, W0, al0, ar0, b0, g0, be0,
           W1, al1, ar1, b1, g1, be1, Wout, bout):
    N, F = x.shape
    H, D = al0.shape
    C = Wout.shape[1]
    E = edge_index_0.shape[1]

    Np = ((N + 64 + _BN - 1) // _BN) * _BN
    egran = 8 * _NSC * _NTILES * _WIN
    Ep = ((E + egran - 1) // egran) * egran
    pad_idx = N + (jnp.arange(Ep - E, dtype=jnp.int32) % 64)

    def pad_edges(ei):
        return (jnp.concatenate([ei[0], pad_idx]),
                jnp.concatenate([ei[1], pad_idx]))

    x_p = jnp.zeros((Np, F), F32).at[:N].set(x)
    src0, dst0 = pad_edges(edge_index_0)
    src1, dst1 = pad_edges(edge_index_1)

    feat0, el0, er0 = _tc_project(x_p, W0.reshape(F, H * D), al0, ar0)
    out0, den0 = _sc_aggregate(src0, dst0, el0.reshape(-1), er0.reshape(-1),
                               feat0.reshape(H * Np, D), Np, D,
                               hps=H // _NSC, edges_all=True)
    val0, st0 = _tc_norm_stats(out0, den0.reshape(H, Np, 1), b0, N,
                               sum_parts=False)

    feat1, el1, er1 = _tc_bn_project(val0, st0, g0.reshape(1, H * D),
                                     be0.reshape(1, H * D), N,
                                     W1.reshape(H * D, D), al1, ar1)
    out1, den1 = _sc_aggregate(src1, dst1, el1.reshape(-1), er1.reshape(-1),
                               feat1.reshape(Np, D), Np, D,
                               hps=1, edges_all=False)
    val1, st1 = _tc_norm_stats(out1, den1.reshape(_NSC, Np, 1), b1, N,
                               sum_parts=True)

    feat_out, logits = _tc_bn_head(val1, st1, g1.reshape(1, D),
                                   be1.reshape(1, D), N, Wout,
                                   bout.reshape(1, C))
    return (logits[:N], feat_out[:N])


# speedup vs baseline: 41.7576x; 1.0203x over previous
import functools

import jax
import jax.numpy as jnp
from jax import lax
from jax.experimental import pallas as pl
from jax.experimental.pallas import tpu as pltpu
from jax.experimental.pallas import tpu_sc as plsc

F32 = jnp.float32
_BN = 256
_WIN = 128
_NSC = 2
_NTILES = 16


def _tc_project(x_p, Wf, al, ar):
    Np, F = x_p.shape
    H, D = al.shape
    nb = Np // _BN
    al3 = al.reshape(H, 1, D)
    ar3 = ar.reshape(H, 1, D)

    def body(x_ref, w_ref, al_ref, ar_ref, f_ref, el_ref, er_ref):
        fb = jnp.dot(x_ref[...], w_ref[...], preferred_element_type=F32)
        f_ref[0] = fb
        el_ref[0] = jnp.sum(fb * al_ref[0], axis=1, keepdims=True)
        er_ref[0] = jnp.sum(fb * ar_ref[0], axis=1, keepdims=True)

    return pl.pallas_call(
        body,
        grid=(H, nb),
        in_specs=[
            pl.BlockSpec((_BN, F), lambda h, i: (i, 0)),
            pl.BlockSpec((F, D), lambda h, i: (0, h)),
            pl.BlockSpec((1, 1, D), lambda h, i: (h, 0, 0)),
            pl.BlockSpec((1, 1, D), lambda h, i: (h, 0, 0)),
        ],
        out_specs=[
            pl.BlockSpec((1, _BN, D), lambda h, i: (h, i, 0)),
            pl.BlockSpec((1, _BN, 1), lambda h, i: (h, i, 0)),
            pl.BlockSpec((1, _BN, 1), lambda h, i: (h, i, 0)),
        ],
        out_shape=[
            jax.ShapeDtypeStruct((H, Np, D), F32),
            jax.ShapeDtypeStruct((H, Np, 1), F32),
            jax.ShapeDtypeStruct((H, Np, 1), F32),
        ],
    )(x_p, Wf, al3, ar3)


def _sc_aggregate(src_p, dst_p, el_f, er_f, feat_f, Np, D, hps, edges_all):
    Ep = src_p.shape[0]
    nslab = _NSC * hps
    rpt = Np // _NTILES
    ept = Ep // _NTILES if edges_all else Ep // (_NTILES * _NSC)
    nwin = ept // _WIN
    assert nwin % 2 == 0
    ch = 16
    ce = ch * _WIN
    assert nwin % ch == 0
    zrows = jnp.zeros((rpt, D), F32)
    zden = jnp.zeros((hps * rpt,), F32)
    mesh = plsc.VectorSubcoreMesh(core_axis_name="c", subcore_axis_name="s")

    @functools.partial(
        pl.kernel,
        out_type=[
            jax.ShapeDtypeStruct((nslab, Np, D), F32),
            jax.ShapeDtypeStruct((nslab, Np), F32),
        ],
        mesh=mesh,
        compiler_params=pltpu.CompilerParams(needs_layout_passes=False),
        scratch_types=[
            pltpu.VMEM_SHARED((Np, D), F32),
            pltpu.VMEM_SHARED((hps * Np,), F32),
            pltpu.VMEM((2, _WIN), jnp.int32),
            pltpu.VMEM((2, _WIN), jnp.int32),
            pltpu.VMEM((2, _WIN), jnp.int32),
            pltpu.VMEM((2, _WIN), jnp.int32),
            pltpu.VMEM((2, _WIN), F32),
            pltpu.VMEM((2, _WIN), F32),
            pltpu.VMEM((2, _WIN), F32),
            pltpu.VMEM((2, _WIN, D), F32),
            pltpu.VMEM((ce,), jnp.int32),
            pltpu.VMEM((ce,), jnp.int32),
            pltpu.SemaphoreType.DMA,
            pltpu.SemaphoreType.DMA,
            pltpu.SemaphoreType.DMA,
            pltpu.SemaphoreType.DMA,
        ],
    )
    def k(src_h, dst_h, el_h, er_h, feat_h, zr_h, zd_h, out_h, den_h,
          out_sp, den_sp, srca_v, dstr_v, srcl_v, dstd_v,
          els_v, erd_v, ee_v, feat_v, srcall_v, dstall_v,
          gsem0, gsem1, ssem0, ssem1):
        c = lax.axis_index("c")
        s = lax.axis_index("s")
        gsem = (gsem0, gsem1)
        ssem = (ssem0, ssem1)
        tile_base = (s * ept) if edges_all else (c * _NTILES + s) * ept
        pltpu.sync_copy(zr_h, out_sp.at[pl.ds(s * rpt, rpt)])
        pltpu.sync_copy(zd_h, den_sp.at[pl.ds(s * (hps * rpt), hps * rpt)])
        plsc.subcore_barrier()
        for p in range(hps):
            head = (c * hps + p) if edges_all else p
            hN = head * Np

            def prefetch(wn, b):
                @pl.when(lax.rem(wn, ch) == 0)
                def _():
                    base = tile_base + wn * _WIN
                    pltpu.sync_copy(src_h.at[pl.ds(base, ce)], srcall_v)
                    pltpu.sync_copy(dst_h.at[pl.ds(base, ce)], dstall_v)

                wofs = lax.rem(wn, ch) * _WIN
                for kk in range(_WIN // 16):
                    sl = pl.ds(kk * 16, 16)
                    lsl = pl.ds(wofs + kk * 16, 16)
                    sv = srcall_v[lsl]
                    dv = dstall_v[lsl]
                    srca_v[b, sl] = sv + hN
                    dstr_v[b, sl] = dv
                    srcl_v[b, sl] = dv + hN
                    dstd_v[b, sl] = dv + (p * Np)
                pltpu.async_copy(el_h.at[srca_v.at[b]], els_v.at[b], gsem[b])
                pltpu.async_copy(er_h.at[srcl_v.at[b]], erd_v.at[b], gsem[b])
                pltpu.async_copy(feat_h.at[srca_v.at[b]], feat_v.at[b],
                                 gsem[b])

            def drain_gather(b):
                pltpu.make_async_copy(
                    el_h.at[pl.ds(0, _WIN)], els_v.at[b], gsem[b]).wait()
                pltpu.make_async_copy(
                    er_h.at[pl.ds(0, _WIN)], erd_v.at[b], gsem[b]).wait()
                pltpu.make_async_copy(
                    feat_h.at[pl.ds(0, _WIN)], feat_v.at[b], gsem[b]).wait()

            def drain_scatter(b):
                pltpu.make_async_copy(
                    ee_v.at[b], den_sp.at[pl.ds(0, _WIN)], ssem[b]).wait()
                pltpu.make_async_copy(
                    feat_v.at[b], out_sp.at[pl.ds(0, _WIN)], ssem[b]).wait()

            def compute(b):
                drain_gather(b)
                for kk in range(_WIN // 16):
                    sl = pl.ds(kk * 16, 16)
                    e = els_v[b, sl] + erd_v[b, sl]
                    e = jnp.where(e >= 0.0, e, 0.2 * e)
                    ee_v[b, sl] = jnp.exp(e)
                pltpu.async_copy(ee_v.at[b], den_sp.at[dstd_v.at[b]],
                                 ssem[b], add=True)

                @pl.loop(0, _WIN, unroll=4)
                def _edge(ei):
                    eev = plsc.load_gather(
                        ee_v.at[b], [jnp.full((16,), ei, jnp.int32)])
                    for k2 in range(D // 16):
                        sl2 = pl.ds(k2 * 16, 16)
                        feat_v[b, ei, sl2] = feat_v[b, ei, sl2] * eev

                pltpu.async_copy(feat_v.at[b], out_sp.at[dstr_v.at[b]],
                                 ssem[b], add=True)

            prefetch(0, 0)

            @pl.loop(0, nwin // 2)
            def _w2(i):
                for b in range(2):
                    w = 2 * i + b
                    b1 = 1 - b

                    @pl.when(w > 0)
                    def _():
                        drain_scatter(b1)

                    prefetch(jnp.minimum(w + 1, nwin - 1), b1)
                    compute(b)

            drain_scatter(1)
            drain_gather(0)
            plsc.subcore_barrier()
            slab = c * hps + p
            pltpu.sync_copy(out_sp.at[pl.ds(s * rpt, rpt)],
                            out_h.at[slab, pl.ds(s * rpt, rpt)])
            if p < hps - 1:
                pltpu.sync_copy(zr_h, out_sp.at[pl.ds(s * rpt, rpt)])
                plsc.subcore_barrier()
        for p in range(hps):
            pltpu.sync_copy(den_sp.at[pl.ds(p * Np + s * rpt, rpt)],
                            den_h.at[c * hps + p, pl.ds(s * rpt, rpt)])

    return k(src_p, dst_p, el_f, er_f, feat_f, zrows, zden)


def _tc_norm_stats(out_slabs, den_slabs, b, n_real, sum_parts):
    S, Np, D = out_slabs.shape
    H = b.shape[0]
    HD = H * D
    nb = Np // _BN

    def body(o_ref, d_ref, b_ref, val_ref, st_ref):
        i = pl.program_id(0)
        if sum_parts:
            acc = o_ref[0]
            den = d_ref[0]
            for t in range(1, S):
                acc = acc + o_ref[t]
                den = den + d_ref[t]
            val = acc / (den + 1e-9) + b_ref[0][None, :]
        else:
            cols = []
            for hh in range(H):
                cols.append(o_ref[hh] / (d_ref[hh] + 1e-9)
                            + b_ref[hh][None, :])
            val = jnp.concatenate(cols, axis=1)
        rows = i * _BN + lax.broadcasted_iota(jnp.int32, (_BN, 1), 0)
        val = jnp.where(rows < n_real, val, 0.0)
        val_ref[...] = val

        @pl.when(i == 0)
        def _():
            st_ref[...] = jnp.zeros_like(st_ref)

        st_ref[0:1, :] += jnp.sum(val, axis=0, keepdims=True)
        st_ref[1:2, :] += jnp.sum(val * val, axis=0, keepdims=True)

    return pl.pallas_call(
        body,
        grid=(nb,),
        in_specs=[
            pl.BlockSpec((S, _BN, D), lambda i: (0, i, 0)),
            pl.BlockSpec((S, _BN, 1), lambda i: (0, i, 0)),
            pl.BlockSpec((H, D), lambda i: (0, 0)),
        ],
        out_specs=[
            pl.BlockSpec((_BN, HD), lambda i: (i, 0)),
            pl.BlockSpec((8, HD), lambda i: (0, 0)),
        ],
        out_shape=[
            jax.ShapeDtypeStruct((Np, HD), F32),
            jax.ShapeDtypeStruct((8, HD), F32),
        ],
    )(out_slabs, den_slabs, b)


def _bn_block(v_ref, st_ref, g_ref, be_ref, i, n_real):
    mean = st_ref[0:1, :] / n_real
    var = st_ref[1:2, :] / n_real - mean * mean
    inv = lax.rsqrt(var + 1e-5)
    hn = (v_ref[...] - mean) * inv * g_ref[...] + be_ref[...]
    hn = jnp.maximum(hn, 0.0)
    rows = i * _BN + lax.broadcasted_iota(jnp.int32, (_BN, 1), 0)
    return jnp.where(rows < n_real, hn, 0.0)


def _tc_bn_project(val, st, g, be, n_real, Wf, al, ar):
    Np, HDin = val.shape
    H, D = al.shape
    nb = Np // _BN
    al3 = al.reshape(H, 1, D)
    ar3 = ar.reshape(H, 1, D)

    def body(v_ref, st_ref, g_ref, be_ref, w_ref, al_ref, ar_ref,
             f_ref, el_ref, er_ref):
        i = pl.program_id(1)
        hb = _bn_block(v_ref, st_ref, g_ref, be_ref, i, n_real)
        fb = jnp.dot(hb, w_ref[...], preferred_element_type=F32)
        f_ref[0] = fb
        el_ref[0] = jnp.sum(fb * al_ref[0], axis=1, keepdims=True)
        er_ref[0] = jnp.sum(fb * ar_ref[0], axis=1, keepdims=True)

    return pl.pallas_call(
        body,
        grid=(H, nb),
        in_specs=[
            pl.BlockSpec((_BN, HDin), lambda h, i: (i, 0)),
            pl.BlockSpec((8, HDin), lambda h, i: (0, 0)),
            pl.BlockSpec((1, HDin), lambda h, i: (0, 0)),
            pl.BlockSpec((1, HDin), lambda h, i: (0, 0)),
            pl.BlockSpec((HDin, D), lambda h, i: (0, h)),
            pl.BlockSpec((1, 1, D), lambda h, i: (h, 0, 0)),
            pl.BlockSpec((1, 1, D), lambda h, i: (h, 0, 0)),
        ],
        out_specs=[
            pl.BlockSpec((1, _BN, D), lambda h, i: (h, i, 0)),
            pl.BlockSpec((1, _BN, 1), lambda h, i: (h, i, 0)),
            pl.BlockSpec((1, _BN, 1), lambda h, i: (h, i, 0)),
        ],
        out_shape=[
            jax.ShapeDtypeStruct((H, Np, D), F32),
            jax.ShapeDtypeStruct((H, Np, 1), F32),
            jax.ShapeDtypeStruct((H, Np, 1), F32),
        ],
    )(val, st, g, be, Wf, al3, ar3)


def _tc_bn_head(val, st, g, be, n_real, Wout, bout):
    Np, D = val.shape
    C = Wout.shape[1]
    nb = Np // _BN

    def body(v_ref, st_ref, g_ref, be_ref, w_ref, bo_ref, f_ref, o_ref):
        i = pl.program_id(0)
        hb = _bn_block(v_ref, st_ref, g_ref, be_ref, i, n_real)
        n2 = jnp.sum(hb * hb, axis=1, keepdims=True)
        nr = jnp.maximum(jnp.sqrt(n2), 1e-12)
        ft = hb / nr
        f_ref[...] = ft
        o_ref[...] = jnp.dot(ft, w_ref[...],
                             preferred_element_type=F32) + bo_ref[...]

    return pl.pallas_call(
        body,
        grid=(nb,),
        in_specs=[
            pl.BlockSpec((_BN, D), lambda i: (i, 0)),
            pl.BlockSpec((8, D), lambda i: (0, 0)),
            pl.BlockSpec((1, D), lambda i: (0, 0)),
            pl.BlockSpec((1, D), lambda i: (0, 0)),
            pl.BlockSpec((D, C), lambda i: (0, 0)),
            pl.BlockSpec((1, C), lambda i: (0, 0)),
        ],
        out_specs=[
            pl.BlockSpec((_BN, D), lambda i: (i, 0)),
            pl.BlockSpec((_BN, C), lambda i: (i, 0)),
        ],
        out_shape=[
            jax.ShapeDtypeStruct((Np, D), F32),
            jax.ShapeDtypeStruct((Np, C), F32),
        ],
    )(val, st, g, be, Wout, bout)


def kernel(x, edge_index_0, edge_index_1, W0, al0, ar0, b0, g0, be0,
           W1, al1, ar1, b1, g1, be1, Wout, bout):
    N, F = x.shape
    H, D = al0.shape
    C = Wout.shape[1]
    E = edge_index_0.shape[1]

    Np = ((N + 64 + _BN - 1) // _BN) * _BN
    egran = 16 * _NSC * _NTILES * _WIN
    Ep = ((E + egran - 1) // egran) * egran
    pad_idx = N + (jnp.arange(Ep - E, dtype=jnp.int32) % 64)

    def pad_edges(ei):
        return (jnp.concatenate([ei[0], pad_idx]),
                jnp.concatenate([ei[1], pad_idx]))

    x_p = jnp.zeros((Np, F), F32).at[:N].set(x)
    src0, dst0 = pad_edges(edge_index_0)
    src1, dst1 = pad_edges(edge_index_1)

    feat0, el0, er0 = _tc_project(x_p, W0.reshape(F, H * D), al0, ar0)
    out0, den0 = _sc_aggregate(src0, dst0, el0.reshape(-1), er0.reshape(-1),
                               feat0.reshape(H * Np, D), Np, D,
                               hps=H // _NSC, edges_all=True)
    val0, st0 = _tc_norm_stats(out0, den0.reshape(H, Np, 1), b0, N,
                               sum_parts=False)

    feat1, el1, er1 = _tc_bn_project(val0, st0, g0.reshape(1, H * D),
                                     be0.reshape(1, H * D), N,
                                     W1.reshape(H * D, D), al1, ar1)
    out1, den1 = _sc_aggregate(src1, dst1, el1.reshape(-1), er1.reshape(-1),
                               feat1.reshape(Np, D), Np, D,
                               hps=1, edges_all=False)
    val1, st1 = _tc_norm_stats(out1, den1.reshape(_NSC, Np, 1), b1, N,
                               sum_parts=True)

    feat_out, logits = _tc_bn_head(val1, st1, g1.reshape(1, D),
                                   be1.reshape(1, D), N, Wout,
                                   bout.reshape(1, C))
    return (logits[:N], feat_out[:N])


# speedup vs baseline: 41.8569x; 1.0024x over previous
import functools

import jax
import jax.numpy as jnp
from jax import lax
from jax.experimental import pallas as pl
from jax.experimental.pallas import tpu as pltpu
from jax.experimental.pallas import tpu_sc as plsc

F32 = jnp.float32
_BN = 256
_WIN = 128
_NSC = 2
_NTILES = 16


def _tc_project(x_p, Wf, al, ar):
    Np, F = x_p.shape
    H, D = al.shape
    nb = Np // _BN
    al3 = al.reshape(H, 1, D)
    ar3 = ar.reshape(H, 1, D)

    def body(x_ref, w_ref, al_ref, ar_ref, f_ref, el_ref, er_ref):
        fb = jnp.dot(x_ref[...], w_ref[...], preferred_element_type=F32)
        f_ref[0] = fb
        el_ref[0] = jnp.sum(fb * al_ref[0], axis=1, keepdims=True)
        er_ref[0] = jnp.sum(fb * ar_ref[0], axis=1, keepdims=True)

    return pl.pallas_call(
        body,
        grid=(H, nb),
        in_specs=[
            pl.BlockSpec((_BN, F), lambda h, i: (i, 0)),
            pl.BlockSpec((F, D), lambda h, i: (0, h)),
            pl.BlockSpec((1, 1, D), lambda h, i: (h, 0, 0)),
            pl.BlockSpec((1, 1, D), lambda h, i: (h, 0, 0)),
        ],
        out_specs=[
            pl.BlockSpec((1, _BN, D), lambda h, i: (h, i, 0)),
            pl.BlockSpec((1, _BN, 1), lambda h, i: (h, i, 0)),
            pl.BlockSpec((1, _BN, 1), lambda h, i: (h, i, 0)),
        ],
        out_shape=[
            jax.ShapeDtypeStruct((H, Np, D), F32),
            jax.ShapeDtypeStruct((H, Np, 1), F32),
            jax.ShapeDtypeStruct((H, Np, 1), F32),
        ],
    )(x_p, Wf, al3, ar3)


def _sc_aggregate(src_p, dst_p, el_f, er_f, feat_f, Np, D, hps, edges_all):
    Ep = src_p.shape[0]
    nslab = _NSC * hps
    rpt = Np // _NTILES
    ept = Ep // _NTILES if edges_all else Ep // (_NTILES * _NSC)
    nwin = ept // _WIN
    assert nwin % 2 == 0
    ch = 16
    ce = ch * _WIN
    assert nwin % ch == 0
    zrows = jnp.zeros((rpt, D), F32)
    zden = jnp.zeros((hps * rpt,), F32)
    mesh = plsc.VectorSubcoreMesh(core_axis_name="c", subcore_axis_name="s")

    @functools.partial(
        pl.kernel,
        out_type=[
            jax.ShapeDtypeStruct((nslab, Np, D), F32),
            jax.ShapeDtypeStruct((nslab, Np), F32),
        ],
        mesh=mesh,
        compiler_params=pltpu.CompilerParams(needs_layout_passes=False),
        scratch_types=[
            pltpu.VMEM_SHARED((Np, D), F32),
            pltpu.VMEM_SHARED((hps * Np,), F32),
            pltpu.VMEM((2, _WIN), jnp.int32),
            pltpu.VMEM((2, _WIN), jnp.int32),
            pltpu.VMEM((2, _WIN), jnp.int32),
            pltpu.VMEM((2, _WIN), jnp.int32),
            pltpu.VMEM((2, _WIN), F32),
            pltpu.VMEM((2, _WIN), F32),
            pltpu.VMEM((2, _WIN), F32),
            pltpu.VMEM((2, _WIN, D), F32),
            pltpu.VMEM((ce,), jnp.int32),
            pltpu.VMEM((ce,), jnp.int32),
            pltpu.SemaphoreType.DMA,
            pltpu.SemaphoreType.DMA,
            pltpu.SemaphoreType.DMA,
            pltpu.SemaphoreType.DMA,
        ],
    )
    def k(src_h, dst_h, el_h, er_h, feat_h, zr_h, zd_h, out_h, den_h,
          out_sp, den_sp, srca_v, dstr_v, srcl_v, dstd_v,
          els_v, erd_v, ee_v, feat_v, srcall_v, dstall_v,
          gsem0, gsem1, ssem0, ssem1):
        c = lax.axis_index("c")
        s = lax.axis_index("s")
        gsem = (gsem0, gsem1)
        ssem = (ssem0, ssem1)
        tile_base = (s * ept) if edges_all else (c * _NTILES + s) * ept
        pltpu.sync_copy(zr_h, out_sp.at[pl.ds(s * rpt, rpt)])
        pltpu.sync_copy(zd_h, den_sp.at[pl.ds(s * (hps * rpt), hps * rpt)])
        plsc.subcore_barrier()
        for p in range(hps):
            head = (c * hps + p) if edges_all else p
            hN = head * Np

            def prefetch(wn, b):
                @pl.when(lax.rem(wn, ch) == 0)
                def _():
                    base = tile_base + wn * _WIN
                    pltpu.sync_copy(src_h.at[pl.ds(base, ce)], srcall_v)
                    pltpu.sync_copy(dst_h.at[pl.ds(base, ce)], dstall_v)

                wofs = lax.rem(wn, ch) * _WIN
                for kk in range(_WIN // 16):
                    sl = pl.ds(kk * 16, 16)
                    lsl = pl.ds(wofs + kk * 16, 16)
                    sv = srcall_v[lsl]
                    dv = dstall_v[lsl]
                    srca_v[b, sl] = sv + hN
                    dstr_v[b, sl] = dv
                    srcl_v[b, sl] = dv + hN
                    dstd_v[b, sl] = dv + (p * Np)
                pltpu.async_copy(el_h.at[srca_v.at[b]], els_v.at[b], gsem[b])
                pltpu.async_copy(er_h.at[srcl_v.at[b]], erd_v.at[b], gsem[b])
                pltpu.async_copy(feat_h.at[srca_v.at[b]], feat_v.at[b],
                                 gsem[b])

            def drain_gather(b):
                pltpu.make_async_copy(
                    el_h.at[pl.ds(0, _WIN)], els_v.at[b], gsem[b]).wait()
                pltpu.make_async_copy(
                    er_h.at[pl.ds(0, _WIN)], erd_v.at[b], gsem[b]).wait()
                pltpu.make_async_copy(
                    feat_h.at[pl.ds(0, _WIN)], feat_v.at[b], gsem[b]).wait()

            def drain_scatter(b):
                pltpu.make_async_copy(
                    ee_v.at[b], den_sp.at[pl.ds(0, _WIN)], ssem[b]).wait()
                pltpu.make_async_copy(
                    feat_v.at[b], out_sp.at[pl.ds(0, _WIN)], ssem[b]).wait()

            def compute(b):
                drain_gather(b)
                for kk in range(_WIN // 16):
                    sl = pl.ds(kk * 16, 16)
                    e = els_v[b, sl] + erd_v[b, sl]
                    e = jnp.where(e >= 0.0, e, 0.2 * e)
                    ee_v[b, sl] = jnp.exp(e)
                pltpu.async_copy(ee_v.at[b], den_sp.at[dstd_v.at[b]],
                                 ssem[b], add=True)

                @pl.loop(0, _WIN, unroll=8)
                def _edge(ei):
                    eev = plsc.load_gather(
                        ee_v.at[b], [jnp.full((16,), ei, jnp.int32)])
                    for k2 in range(D // 16):
                        sl2 = pl.ds(k2 * 16, 16)
                        feat_v[b, ei, sl2] = feat_v[b, ei, sl2] * eev

                pltpu.async_copy(feat_v.at[b], out_sp.at[dstr_v.at[b]],
                                 ssem[b], add=True)

            prefetch(0, 0)

            @pl.loop(0, nwin // 2)
            def _w2(i):
                for b in range(2):
                    w = 2 * i + b
                    b1 = 1 - b

                    @pl.when(w > 0)
                    def _():
                        drain_scatter(b1)

                    prefetch(jnp.minimum(w + 1, nwin - 1), b1)
                    compute(b)

            drain_scatter(1)
            drain_gather(0)
            plsc.subcore_barrier()
            slab = c * hps + p
            pltpu.sync_copy(out_sp.at[pl.ds(s * rpt, rpt)],
                            out_h.at[slab, pl.ds(s * rpt, rpt)])
            if p < hps - 1:
                pltpu.sync_copy(zr_h, out_sp.at[pl.ds(s * rpt, rpt)])
                plsc.subcore_barrier()
        for p in range(hps):
            pltpu.sync_copy(den_sp.at[pl.ds(p * Np + s * rpt, rpt)],
                            den_h.at[c * hps + p, pl.ds(s * rpt, rpt)])

    return k(src_p, dst_p, el_f, er_f, feat_f, zrows, zden)


def _tc_norm_stats(out_slabs, den_slabs, b, n_real, sum_parts):
    S, Np, D = out_slabs.shape
    H = b.shape[0]
    HD = H * D
    nb = Np // _BN

    def body(o_ref, d_ref, b_ref, val_ref, st_ref):
        i = pl.program_id(0)
        if sum_parts:
            acc = o_ref[0]
            den = d_ref[0]
            for t in range(1, S):
                acc = acc + o_ref[t]
                den = den + d_ref[t]
            val = acc / (den + 1e-9) + b_ref[0][None, :]
        else:
            cols = []
            for hh in range(H):
                cols.append(o_ref[hh] / (d_ref[hh] + 1e-9)
                            + b_ref[hh][None, :])
            val = jnp.concatenate(cols, axis=1)
        rows = i * _BN + lax.broadcasted_iota(jnp.int32, (_BN, 1), 0)
        val = jnp.where(rows < n_real, val, 0.0)
        val_ref[...] = val

        @pl.when(i == 0)
        def _():
            st_ref[...] = jnp.zeros_like(st_ref)

        st_ref[0:1, :] += jnp.sum(val, axis=0, keepdims=True)
        st_ref[1:2, :] += jnp.sum(val * val, axis=0, keepdims=True)

    return pl.pallas_call(
        body,
        grid=(nb,),
        in_specs=[
            pl.BlockSpec((S, _BN, D), lambda i: (0, i, 0)),
            pl.BlockSpec((S, _BN, 1), lambda i: (0, i, 0)),
            pl.BlockSpec((H, D), lambda i: (0, 0)),
        ],
        out_specs=[
            pl.BlockSpec((_BN, HD), lambda i: (i, 0)),
            pl.BlockSpec((8, HD), lambda i: (0, 0)),
        ],
        out_shape=[
            jax.ShapeDtypeStruct((Np, HD), F32),
            jax.ShapeDtypeStruct((8, HD), F32),
        ],
    )(out_slabs, den_slabs, b)


def _bn_block(v_ref, st_ref, g_ref, be_ref, i, n_real):
    mean = st_ref[0:1, :] / n_real
    var = st_ref[1:2, :] / n_real - mean * mean
    inv = lax.rsqrt(var + 1e-5)
    hn = (v_ref[...] - mean) * inv * g_ref[...] + be_ref[...]
    hn = jnp.maximum(hn, 0.0)
    rows = i * _BN + lax.broadcasted_iota(jnp.int32, (_BN, 1), 0)
    return jnp.where(rows < n_real, hn, 0.0)


def _tc_bn_project(val, st, g, be, n_real, Wf, al, ar):
    Np, HDin = val.shape
    H, D = al.shape
    nb = Np // _BN
    al3 = al.reshape(H, 1, D)
    ar3 = ar.reshape(H, 1, D)

    def body(v_ref, st_ref, g_ref, be_ref, w_ref, al_ref, ar_ref,
             f_ref, el_ref, er_ref):
        i = pl.program_id(1)
        hb = _bn_block(v_ref, st_ref, g_ref, be_ref, i, n_real)
        fb = jnp.dot(hb, w_ref[...], preferred_element_type=F32)
        f_ref[0] = fb
        el_ref[0] = jnp.sum(fb * al_ref[0], axis=1, keepdims=True)
        er_ref[0] = jnp.sum(fb * ar_ref[0], axis=1, keepdims=True)

    return pl.pallas_call(
        body,
        grid=(H, nb),
        in_specs=[
            pl.BlockSpec((_BN, HDin), lambda h, i: (i, 0)),
            pl.BlockSpec((8, HDin), lambda h, i: (0, 0)),
            pl.BlockSpec((1, HDin), lambda h, i: (0, 0)),
            pl.BlockSpec((1, HDin), lambda h, i: (0, 0)),
            pl.BlockSpec((HDin, D), lambda h, i: (0, h)),
            pl.BlockSpec((1, 1, D), lambda h, i: (h, 0, 0)),
            pl.BlockSpec((1, 1, D), lambda h, i: (h, 0, 0)),
        ],
        out_specs=[
            pl.BlockSpec((1, _BN, D), lambda h, i: (h, i, 0)),
            pl.BlockSpec((1, _BN, 1), lambda h, i: (h, i, 0)),
            pl.BlockSpec((1, _BN, 1), lambda h, i: (h, i, 0)),
        ],
        out_shape=[
            jax.ShapeDtypeStruct((H, Np, D), F32),
            jax.ShapeDtypeStruct((H, Np, 1), F32),
            jax.ShapeDtypeStruct((H, Np, 1), F32),
        ],
    )(val, st, g, be, Wf, al3, ar3)


def _tc_bn_head(val, st, g, be, n_real, Wout, bout):
    Np, D = val.shape
    C = Wout.shape[1]
    nb = Np // _BN

    def body(v_ref, st_ref, g_ref, be_ref, w_ref, bo_ref, f_ref, o_ref):
        i = pl.program_id(0)
        hb = _bn_block(v_ref, st_ref, g_ref, be_ref, i, n_real)
        n2 = jnp.sum(hb * hb, axis=1, keepdims=True)
        nr = jnp.maximum(jnp.sqrt(n2), 1e-12)
        ft = hb / nr
        f_ref[...] = ft
        o_ref[...] = jnp.dot(ft, w_ref[...],
                             preferred_element_type=F32) + bo_ref[...]

    return pl.pallas_call(
        body,
        grid=(nb,),
        in_specs=[
            pl.BlockSpec((_BN, D), lambda i: (i, 0)),
            pl.BlockSpec((8, D), lambda i: (0, 0)),
            pl.BlockSpec((1, D), lambda i: (0, 0)),
            pl.BlockSpec((1, D), lambda i: (0, 0)),
            pl.BlockSpec((D, C), lambda i: (0, 0)),
            pl.BlockSpec((1, C), lambda i: (0, 0)),
        ],
        out_specs=[
            pl.BlockSpec((_BN, D), lambda i: (i, 0)),
            pl.BlockSpec((_BN, C), lambda i: (i, 0)),
        ],
        out_shape=[
            jax.ShapeDtypeStruct((Np, D), F32),
            jax.ShapeDtypeStruct((Np, C), F32),
        ],
    )(val, st, g, be, Wout, bout)


def kernel(x, edge_index_0, edge_index_1, W0, al0, ar0, b0, g0, be0,
           W1, al1, ar1, b1, g1, be1, Wout, bout):
    N, F = x.shape
    H, D = al0.shape
    C = Wout.shape[1]
    E = edge_index_0.shape[1]

    Np = ((N + 64 + _BN - 1) // _BN) * _BN
    egran = 16 * _NSC * _NTILES * _WIN
    Ep = ((E + egran - 1) // egran) * egran
    pad_idx = N + (jnp.arange(Ep - E, dtype=jnp.int32) % 64)

    def pad_edges(ei):
        return (jnp.concatenate([ei[0], pad_idx]),
                jnp.concatenate([ei[1], pad_idx]))

    x_p = jnp.zeros((Np, F), F32).at[:N].set(x)
    src0, dst0 = pad_edges(edge_index_0)
    src1, dst1 = pad_edges(edge_index_1)

    feat0, el0, er0 = _tc_project(x_p, W0.reshape(F, H * D), al0, ar0)
    out0, den0 = _sc_aggregate(src0, dst0, el0.reshape(-1), er0.reshape(-1),
                               feat0.reshape(H * Np, D), Np, D,
                               hps=H // _NSC, edges_all=True)
    val0, st0 = _tc_norm_stats(out0, den0.reshape(H, Np, 1), b0, N,
                               sum_parts=False)

    feat1, el1, er1 = _tc_bn_project(val0, st0, g0.reshape(1, H * D),
                                     be0.reshape(1, H * D), N,
                                     W1.reshape(H * D, D), al1, ar1)
    out1, den1 = _sc_aggregate(src1, dst1, el1.reshape(-1), er1.reshape(-1),
                               feat1.reshape(Np, D), Np, D,
                               hps=1, edges_all=False)
    val1, st1 = _tc_norm_stats(out1, den1.reshape(_NSC, Np, 1), b1, N,
                               sum_parts=True)

    feat_out, logits = _tc_bn_head(val1, st1, g1.reshape(1, D),
                                   be1.reshape(1, D), N, Wout,
                                   bout.reshape(1, C))
    return (logits[:N], feat_out[:N])
